# Initial kernel scaffold; baseline (speedup 1.0000x reference)
import jax
import jax.numpy as jnp
from jax.experimental import pallas as pl


def kernel(x, edge_index, batch, W1, b1, Wf1, bf1, g1, be1, Ws1, bs1, Wr1, W2, b2, Wf2, bf2, g2, be2, Ws2, bs2, Wr2, W3, b3, Wf3, bf3, g3, be3, Ws3, bs3, Wr3):
    raise NotImplementedError("write your pallas kernel here")


# speedup vs baseline: 36.7950x; 36.7950x over previous
import functools
import numpy as np
import jax
import jax.numpy as jnp
from jax import lax
from jax.experimental import pallas as pl
from jax.experimental.pallas import tpu as pltpu
from jax.experimental.pallas import tpu_sc as plsc

NND = 10000
NROWS = 10240
NTRASH = NROWS - NND
E = 640000
H = 128
NC, NS = 2, 16
NT = NC * NS
WIN = 128
CAPW = 20480
NWIN_T = CAPW // WIN
NTE = NT * CAPW
ROWS_S = NROWS // NS
GSCALE = float(1.0 / np.sqrt(1.0 + 1e-5))
MINKEY = np.int32(-2**31)

_MESH = plsc.VectorSubcoreMesh(core_axis_name="c", subcore_axis_name="s")


DGRP = 4


def _deg_body(dst_hbm, zer1_hbm, degp_hbm, didx, ones_v, acc_sh, sem_i, sem_s):
    c = lax.axis_index("c")
    s = lax.axis_index("s")
    wid = s * NC + c
    for i in range(WIN // 16):
        ones_v[pl.ds(i * 16, 16)] = jnp.ones((16,), jnp.float32)
    pltpu.sync_copy(zer1_hbm.at[pl.ds(s * ROWS_S, ROWS_S)],
                    acc_sh.at[pl.ds(s * ROWS_S, ROWS_S)])
    plsc.subcore_barrier()

    def body(i, carry):
        base = wid * CAPW + i * (DGRP * WIN)
        cps = [pltpu.async_copy(dst_hbm.at[pl.ds(base + j * WIN, WIN)],
                                didx.at[j], sem_i) for j in range(DGRP)]
        for cp in cps:
            cp.wait()
        sc_ = [pltpu.async_copy(ones_v, acc_sh.at[didx.at[j]], sem_s, add=True)
               for j in range(DGRP)]
        for cp in sc_:
            cp.wait()
        return carry

    lax.fori_loop(0, NWIN_T // DGRP, body, 0)
    plsc.subcore_barrier()
    pltpu.sync_copy(acc_sh.at[pl.ds(s * ROWS_S, ROWS_S)],
                    degp_hbm.at[c, pl.ds(s * ROWS_S, ROWS_S)])


_deg_call = pl.kernel(
    _deg_body,
    out_type=jax.ShapeDtypeStruct((NC, NROWS), jnp.float32),
    mesh=_MESH,
    scratch_types=[
        pltpu.VMEM((DGRP, WIN), jnp.int32),
        pltpu.VMEM((WIN,), jnp.float32),
        pltpu.VMEM_SHARED((NROWS,), jnp.float32),
        pltpu.SemaphoreType.DMA,
        pltpu.SemaphoreType.DMA,
    ],
)


AGRP = 2


def _agg_body(hp_hbm, src_hbm, dst_hbm, zer2_hbm, out_hbm,
              sidx, didx, rows, acc_sh, sem_i, sem_g, sem_s):
    c = lax.axis_index("c")
    s = lax.axis_index("s")
    wid = s * NC + c
    pltpu.sync_copy(zer2_hbm.at[pl.ds(s * ROWS_S, ROWS_S)],
                    acc_sh.at[pl.ds(s * ROWS_S, ROWS_S)])
    plsc.subcore_barrier()

    def body(i, carry):
        base = wid * CAPW + i * (AGRP * WIN)
        cp0 = pltpu.async_copy(src_hbm.at[pl.ds(base, AGRP * WIN)], sidx, sem_i)
        cps = [pltpu.async_copy(dst_hbm.at[pl.ds(base + j * WIN, WIN)],
                                didx.at[j], sem_i) for j in range(AGRP)]
        cp0.wait()
        for cp in cps:
            cp.wait()
        gs = [pltpu.async_copy(hp_hbm.at[sidx.at[pl.ds(j * WIN, WIN)]],
                               rows.at[j], sem_g) for j in range(AGRP)]
        for cp in gs:
            cp.wait()
        ss = [pltpu.async_copy(rows.at[j], acc_sh.at[didx.at[j]], sem_s,
                               add=True) for j in range(AGRP)]
        for cp in ss:
            cp.wait()
        return carry

    lax.fori_loop(0, NWIN_T // AGRP, body, 0)
    plsc.subcore_barrier()
    pltpu.sync_copy(acc_sh.at[pl.ds(s * ROWS_S, ROWS_S)],
                    out_hbm.at[c, pl.ds(s * ROWS_S, ROWS_S)])


_agg_call = pl.kernel(
    _agg_body,
    out_type=jax.ShapeDtypeStruct((NC, NROWS, H), jnp.float32),
    mesh=_MESH,
    scratch_types=[
        pltpu.VMEM((AGRP * WIN,), jnp.int32),
        pltpu.VMEM((AGRP, WIN), jnp.int32),
        pltpu.VMEM((AGRP, WIN, H), jnp.float32),
        pltpu.VMEM_SHARED((NROWS, H), jnp.float32),
        pltpu.SemaphoreType.DMA,
        pltpu.SemaphoreType.DMA,
        pltpu.SemaphoreType.DMA,
    ],
)


SGRP = 4


def _ssc_body(sn_hbm, src_hbm, dst_hbm, zer1_hbm, out_hbm,
              sidx, didx, vals, acc_sh, sem_i, sem_g, sem_s):
    c = lax.axis_index("c")
    s = lax.axis_index("s")
    wid = s * NC + c
    pltpu.sync_copy(zer1_hbm.at[pl.ds(s * ROWS_S, ROWS_S)],
                    acc_sh.at[pl.ds(s * ROWS_S, ROWS_S)])
    plsc.subcore_barrier()

    def body(i, carry):
        base = wid * CAPW + i * (SGRP * WIN)
        cp0 = pltpu.async_copy(src_hbm.at[pl.ds(base, SGRP * WIN)], sidx, sem_i)
        cps = [pltpu.async_copy(dst_hbm.at[pl.ds(base + j * WIN, WIN)],
                                didx.at[j], sem_i) for j in range(SGRP)]
        cp0.wait()
        for cp in cps:
            cp.wait()
        gs = [pltpu.async_copy(sn_hbm.at[sidx.at[pl.ds(j * WIN, WIN)]],
                               vals.at[j], sem_g) for j in range(SGRP)]
        for cp in gs:
            cp.wait()
        ss = [pltpu.async_copy(vals.at[j], acc_sh.at[didx.at[j]], sem_s,
                               add=True) for j in range(SGRP)]
        for cp in ss:
            cp.wait()
        return carry

    lax.fori_loop(0, NWIN_T // SGRP, body, 0)
    plsc.subcore_barrier()
    pltpu.sync_copy(acc_sh.at[pl.ds(s * ROWS_S, ROWS_S)],
                    out_hbm.at[c, pl.ds(s * ROWS_S, ROWS_S)])


_ssc_call = pl.kernel(
    _ssc_body,
    out_type=jax.ShapeDtypeStruct((NC, NROWS), jnp.float32),
    mesh=_MESH,
    scratch_types=[
        pltpu.VMEM((SGRP * WIN,), jnp.int32),
        pltpu.VMEM((SGRP, WIN), jnp.int32),
        pltpu.VMEM((SGRP, WIN), jnp.float32),
        pltpu.VMEM_SHARED((NROWS,), jnp.float32),
        pltpu.SemaphoreType.DMA,
        pltpu.SemaphoreType.DMA,
        pltpu.SemaphoreType.DMA,
    ],
)


EGRP = 4


def _edg_body(kept_hbm, src_hbm, dst_hbm, srco_hbm, dsto_hbm,
              sbuf, dbuf, ksv, kdv, sem_i, sem_g):
    c = lax.axis_index("c")
    s = lax.axis_index("s")
    wid = s * NC + c
    lane = lax.iota(jnp.int32, 16)

    def body(i, carry):
        base = wid * CAPW + i * (EGRP * WIN)
        cp0 = pltpu.async_copy(src_hbm.at[pl.ds(base, EGRP * WIN)], sbuf, sem_i)
        cp1 = pltpu.async_copy(dst_hbm.at[pl.ds(base, EGRP * WIN)], dbuf, sem_i)
        cp0.wait()
        cp1.wait()
        gs = [pltpu.async_copy(kept_hbm.at[sbuf.at[pl.ds(j * WIN, WIN)]],
                               ksv.at[j], sem_g) for j in range(EGRP)]
        gd = [pltpu.async_copy(kept_hbm.at[dbuf.at[pl.ds(j * WIN, WIN)]],
                               kdv.at[j], sem_g) for j in range(EGRP)]
        for cp in gs + gd:
            cp.wait()

        for j in range(EGRP):
            def grp_body(q, carry2, j=j):
                off = j * WIN + q * 16
                s_v = sbuf[pl.ds(off, 16)]
                d_v = dbuf[pl.ds(off, 16)]
                ks = ksv[j, pl.ds(q * 16, 16)]
                kd = kdv[j, pl.ds(q * 16, 16)]
                live = (ks * kd) > 0.5
                spread = wid * 577 + i * 131 + off + lane
                sbuf[pl.ds(off, 16)] = jnp.where(live, s_v, spread % NND)
                dbuf[pl.ds(off, 16)] = jnp.where(live, d_v,
                                                 NND + (spread % NTRASH))
                return carry2

            lax.fori_loop(0, WIN // 16, grp_body, 0)
        cp2 = pltpu.async_copy(sbuf, srco_hbm.at[pl.ds(base, EGRP * WIN)], sem_i)
        cp3 = pltpu.async_copy(dbuf, dsto_hbm.at[pl.ds(base, EGRP * WIN)], sem_i)
        cp2.wait()
        cp3.wait()
        return carry

    lax.fori_loop(0, NWIN_T // EGRP, body, 0)


_edg_call = pl.kernel(
    _edg_body,
    out_type=[
        jax.ShapeDtypeStruct((NTE,), jnp.int32),
        jax.ShapeDtypeStruct((NTE,), jnp.int32),
    ],
    mesh=_MESH,
    scratch_types=[
        pltpu.VMEM((EGRP * WIN,), jnp.int32),
        pltpu.VMEM((EGRP * WIN,), jnp.int32),
        pltpu.VMEM((EGRP, WIN), jnp.float32),
        pltpu.VMEM((EGRP, WIN), jnp.float32),
        pltpu.SemaphoreType.DMA,
        pltpu.SemaphoreType.DMA,
    ],
)


def _prep_body(W1, Wf1, b1, bf1, W2, Wf2, b2, bf2, W3, Wf3, b3, bf3,
               Wc1, bc1, Wc2, bc2, Wc3, bc3):
    hi = jax.lax.Precision.HIGHEST
    Wc1[...] = jnp.dot(W1[...], Wf1[...], precision=hi)
    bc1[...] = jnp.dot(b1[...], Wf1[...], precision=hi) + bf1[...]
    Wc2[...] = jnp.dot(W2[...], Wf2[...], precision=hi)
    bc2[...] = jnp.dot(b2[...], Wf2[...], precision=hi) + bf2[...]
    Wc3[...] = jnp.dot(W3[...], Wf3[...], precision=hi)
    bc3[...] = jnp.dot(b3[...], Wf3[...], precision=hi) + bf3[...]


def _prep_call(W1, Wf1, b1, bf1, W2, Wf2, b2, bf2, W3, Wf3, b3, bf3):
    DIN = W1.shape[0]
    return pl.pallas_call(
        _prep_body,
        out_shape=[
            jax.ShapeDtypeStruct((DIN, H), jnp.float32),
            jax.ShapeDtypeStruct((1, H), jnp.float32),
            jax.ShapeDtypeStruct((H, H), jnp.float32),
            jax.ShapeDtypeStruct((1, H), jnp.float32),
            jax.ShapeDtypeStruct((H, H), jnp.float32),
            jax.ShapeDtypeStruct((1, H), jnp.float32),
        ],
    )(W1, Wf1, b1, bf1, W2, Wf2, b2, bf2, W3, Wf3, b3, bf3)


NBLK = 8
RBLK = NROWS // NBLK


def _lin_body(x, Wc, degp, hp, dinv):
    d = degp[0] + degp[1] + 1.0
    di = lax.rsqrt(d)
    h = jnp.dot(x[...], Wc[...], precision=jax.lax.Precision.HIGHEST)
    hp[...] = h * di
    dinv[...] = di


def _lin_call(x, Wc, degp):
    DIN = x.shape[1]
    return pl.pallas_call(
        _lin_body,
        grid=(NBLK,),
        in_specs=[
            pl.BlockSpec((RBLK, DIN), lambda r: (r, 0)),
            pl.BlockSpec((DIN, H), lambda r: (0, 0)),
            pl.BlockSpec((NC, RBLK, 1), lambda r: (0, r, 0)),
        ],
        out_specs=[
            pl.BlockSpec((RBLK, H), lambda r: (r, 0)),
            pl.BlockSpec((RBLK, 1), lambda r: (r, 0)),
        ],
        out_shape=[
            jax.ShapeDtypeStruct((NROWS, H), jnp.float32),
            jax.ShapeDtypeStruct((NROWS, 1), jnp.float32),
        ],
    )(x, Wc, degp)


def _post_body(aggp, hp, dinv, bc, g, be, Wsr, xl, sn, rn):
    y = dinv[...] * (aggp[0] + aggp[1] + hp[...]) + bc[...]
    xv = jnp.maximum(y, 0.0) * (g[...] * GSCALE) + be[...]
    xl[...] = xv
    sr = jnp.dot(xv, Wsr[...], precision=jax.lax.Precision.HIGHEST)
    sn[...] = sr[:, 0:1]
    rn[...] = sr[:, 1:2]


def _post_call(aggp, hp, dinv, bc, g, be, Wsr):
    return pl.pallas_call(
        _post_body,
        grid=(NBLK,),
        in_specs=[
            pl.BlockSpec((NC, RBLK, H), lambda r: (0, r, 0)),
            pl.BlockSpec((RBLK, H), lambda r: (r, 0)),
            pl.BlockSpec((RBLK, 1), lambda r: (r, 0)),
            pl.BlockSpec((1, H), lambda r: (0, 0)),
            pl.BlockSpec((1, H), lambda r: (0, 0)),
            pl.BlockSpec((1, H), lambda r: (0, 0)),
            pl.BlockSpec((H, 2), lambda r: (0, 0)),
        ],
        out_specs=[
            pl.BlockSpec((RBLK, H), lambda r: (r, 0)),
            pl.BlockSpec((RBLK, 1), lambda r: (r, 0)),
            pl.BlockSpec((RBLK, 1), lambda r: (r, 0)),
        ],
        out_shape=[
            jax.ShapeDtypeStruct((NROWS, H), jnp.float32),
            jax.ShapeDtypeStruct((NROWS, 1), jnp.float32),
            jax.ShapeDtypeStruct((NROWS, 1), jnp.float32),
        ],
    )(aggp, hp, dinv, bc, g, be, Wsr)


def _pool_body(k, final, saggp, rn, bs, xl, alive, *outs):
    sc = saggp[0] + saggp[1] + bs[0, 0] + rn[...]
    u = lax.bitcast_convert_type(sc, jnp.int32)
    key = jnp.where(u < 0, (~u) ^ MINKEY, u)
    key = jnp.where(alive[...] > 0.5, key, MINKEY)

    def kth_body(i, lohi):
        lo, hi = lohi
        mid = (lo >> 1) + (hi >> 1) + (lo & hi & 1)
        cnt = jnp.sum((key >= mid).astype(jnp.int32))
        good = cnt >= k
        return jnp.where(good, mid, lo), jnp.where(good, hi, mid)

    t, _ = lax.fori_loop(0, 33, kth_body,
                         (jnp.full((), MINKEY), jnp.full((), np.int32(2**31 - 1))))
    cnt_gt = jnp.sum((key > t).astype(jnp.int32))
    tneed = k - cnt_gt
    idx = lax.broadcasted_iota(jnp.int32, key.shape, 0)
    ties = key == t

    def tie_body(i, lohi):
        lo, hi = lohi
        mid = (lo + hi) // 2
        cnt = jnp.sum((ties & (idx <= mid)).astype(jnp.int32))
        good = cnt >= tneed
        return jnp.where(good, lo, mid), jnp.where(good, mid, hi)

    _, m = lax.fori_loop(0, 15, tie_body,
                         (jnp.full((), np.int32(-1)), jnp.full((), np.int32(NROWS - 1))))
    kept = ((key > t) | (ties & (idx <= m))).astype(jnp.float32)
    tsc = jnp.tanh(sc) * kept
    xp = xl[...] * tsc
    if final:
        outs[0][...] = jnp.sum(xp, axis=0, keepdims=True) * (1.0 / float(k))
    else:
        outs[0][...] = xp
        outs[1][...] = kept


def _pool_call(k, final, saggp, rn, bs, xl, alive):
    if final:
        out_shape = [jax.ShapeDtypeStruct((1, H), jnp.float32)]
    else:
        out_shape = [jax.ShapeDtypeStruct((NROWS, H), jnp.float32),
                     jax.ShapeDtypeStruct((NROWS, 1), jnp.float32)]
    return pl.pallas_call(
        functools.partial(_pool_body, k, final),
        out_shape=out_shape,
    )(saggp, rn, bs, xl, alive)


def kernel(x, edge_index, batch,
           W1, b1, Wf1, bf1, g1, be1, Ws1, bs1, Wr1,
           W2, b2, Wf2, bf2, g2, be2, Ws2, bs2, Wr2,
           W3, b3, Wf3, bf3, g3, be3, Ws3, bs3, Wr3):
    f32 = jnp.float32
    x = x.astype(f32)
    src = edge_index[0].astype(jnp.int32).reshape(NT, E // NT)
    dst = edge_index[1].astype(jnp.int32).reshape(NT, E // NT)
    npad = CAPW - E // NT
    t_i = jnp.arange(NT, dtype=jnp.int32)[:, None]
    j_i = jnp.arange(npad, dtype=jnp.int32)[None, :]
    pad_s = (t_i * 313 + j_i) % NND
    pad_d = NND + (t_i * 37 + j_i) % NTRASH
    srcC = jnp.concatenate([src, pad_s], axis=1).reshape(-1)
    dstC = jnp.concatenate([dst, pad_d], axis=1).reshape(-1)

    xp = jnp.zeros((NROWS, x.shape[1]), f32).at[:NND].set(x)
    alive = (jnp.arange(NROWS) < NND).astype(f32).reshape(NROWS, 1)
    zer1 = jnp.zeros((NROWS,), f32)
    zer2 = jnp.zeros((NROWS, H), f32)

    Wc1, bc1, Wc2, bc2, Wc3, bc3 = _prep_call(
        W1, Wf1, b1.reshape(1, H), bf1.reshape(1, H),
        W2, Wf2, b2.reshape(1, H), bf2.reshape(1, H),
        W3, Wf3, b3.reshape(1, H), bf3.reshape(1, H))

    layer_params = [
        (Wc1, bc1, g1, be1, Ws1, bs1, Wr1, 5000),
        (Wc2, bc2, g2, be2, Ws2, bs2, Wr2, 2500),
        (Wc3, bc3, g3, be3, Ws3, bs3, Wr3, 1250),
    ]

    for li, (Wc, bc, g, be, Ws, bs, Wr, k) in enumerate(layer_params):
        final = li == 2
        degp = _deg_call(dstC, zer1)
        hp, dinv = _lin_call(xp, Wc, degp.reshape(NC, NROWS, 1))
        aggp = _agg_call(hp, srcC, dstC, zer2)
        xl, sn, rn = _post_call(aggp, hp, dinv, bc, g.reshape(1, H),
                                be.reshape(1, H),
                                jnp.concatenate([Ws, Wr], axis=1))
        saggp = _ssc_call(sn.reshape(NROWS), srcC, dstC, zer1)
        outs = _pool_call(k, final, saggp.reshape(NC, NROWS, 1), rn,
                          bs.reshape(1, 1), xl, alive)
        if final:
            return outs[0]
        xp, kept = outs
        srcC, dstC = _edg_call(kept.reshape(NROWS), srcC, dstC)
        alive = kept


# speedup vs baseline: 38.4158x; 1.0440x over previous
import functools
import numpy as np
import jax
import jax.numpy as jnp
from jax import lax
from jax.experimental import pallas as pl
from jax.experimental.pallas import tpu as pltpu
from jax.experimental.pallas import tpu_sc as plsc

NND = 10000
NROWS = 10240
NTRASH = NROWS - NND
E = 640000
H = 128
NC, NS = 2, 16
NT = NC * NS
WIN = 128
CAPW = 20480
NWIN_T = CAPW // WIN
NTE = NT * CAPW
ROWS_S = NROWS // NS
GSCALE = float(1.0 / np.sqrt(1.0 + 1e-5))
MINKEY = np.int32(-2**31)

_MESH = plsc.VectorSubcoreMesh(core_axis_name="c", subcore_axis_name="s")


DGRP = 4


def _deg_body(dst_hbm, zer1_hbm, degp_hbm, didx, ones_v, acc_sh, sem_i, sem_s):
    c = lax.axis_index("c")
    s = lax.axis_index("s")
    wid = s * NC + c
    for i in range(WIN // 16):
        ones_v[pl.ds(i * 16, 16)] = jnp.ones((16,), jnp.float32)
    pltpu.sync_copy(zer1_hbm.at[pl.ds(s * ROWS_S, ROWS_S)],
                    acc_sh.at[pl.ds(s * ROWS_S, ROWS_S)])
    plsc.subcore_barrier()

    def body(i, carry):
        base = wid * CAPW + i * (DGRP * WIN)
        cps = [pltpu.async_copy(dst_hbm.at[pl.ds(base + j * WIN, WIN)],
                                didx.at[j], sem_i) for j in range(DGRP)]
        sc_ = []
        for j in range(DGRP):
            cps[j].wait()
            sc_.append(pltpu.async_copy(ones_v, acc_sh.at[didx.at[j]],
                                        sem_s, add=True))
        for cp in sc_:
            cp.wait()
        return carry

    lax.fori_loop(0, NWIN_T // DGRP, body, 0)
    plsc.subcore_barrier()
    pltpu.sync_copy(acc_sh.at[pl.ds(s * ROWS_S, ROWS_S)],
                    degp_hbm.at[c, pl.ds(s * ROWS_S, ROWS_S)])


_deg_call = pl.kernel(
    _deg_body,
    out_type=jax.ShapeDtypeStruct((NC, NROWS), jnp.float32),
    mesh=_MESH,
    scratch_types=[
        pltpu.VMEM((DGRP, WIN), jnp.int32),
        pltpu.VMEM((WIN,), jnp.float32),
        pltpu.VMEM_SHARED((NROWS,), jnp.float32),
        pltpu.SemaphoreType.DMA,
        pltpu.SemaphoreType.DMA,
    ],
)


AGRP = 2


def _agg_body(hp_hbm, src_hbm, dst_hbm, zer2_hbm, out_hbm,
              sidx, didx, rows, acc_sh, sem_i, sem_g, sem_s):
    c = lax.axis_index("c")
    s = lax.axis_index("s")
    wid = s * NC + c
    pltpu.sync_copy(zer2_hbm.at[pl.ds(s * ROWS_S, ROWS_S)],
                    acc_sh.at[pl.ds(s * ROWS_S, ROWS_S)])
    plsc.subcore_barrier()

    def body(i, carry):
        base = wid * CAPW + i * (AGRP * WIN)
        cp0 = pltpu.async_copy(src_hbm.at[pl.ds(base, AGRP * WIN)], sidx, sem_i)
        cps = [pltpu.async_copy(dst_hbm.at[pl.ds(base + j * WIN, WIN)],
                                didx.at[j], sem_i) for j in range(AGRP)]
        cp0.wait()
        gs = [pltpu.async_copy(hp_hbm.at[sidx.at[pl.ds(j * WIN, WIN)]],
                               rows.at[j], sem_g) for j in range(AGRP)]
        ss = []
        for j in range(AGRP):
            cps[j].wait()
            gs[j].wait()
            ss.append(pltpu.async_copy(rows.at[j], acc_sh.at[didx.at[j]],
                                       sem_s, add=True))
        for cp in ss:
            cp.wait()
        return carry

    lax.fori_loop(0, NWIN_T // AGRP, body, 0)
    plsc.subcore_barrier()
    pltpu.sync_copy(acc_sh.at[pl.ds(s * ROWS_S, ROWS_S)],
                    out_hbm.at[c, pl.ds(s * ROWS_S, ROWS_S)])


_agg_call = pl.kernel(
    _agg_body,
    out_type=jax.ShapeDtypeStruct((NC, NROWS, H), jnp.float32),
    mesh=_MESH,
    scratch_types=[
        pltpu.VMEM((AGRP * WIN,), jnp.int32),
        pltpu.VMEM((AGRP, WIN), jnp.int32),
        pltpu.VMEM((AGRP, WIN, H), jnp.float32),
        pltpu.VMEM_SHARED((NROWS, H), jnp.float32),
        pltpu.SemaphoreType.DMA,
        pltpu.SemaphoreType.DMA,
        pltpu.SemaphoreType.DMA,
    ],
)


SGRP = 4


def _ssc_body(sn_hbm, src_hbm, dst_hbm, zer1_hbm, out_hbm,
              sidx, didx, vals, acc_sh, sem_i, sem_g, sem_s):
    c = lax.axis_index("c")
    s = lax.axis_index("s")
    wid = s * NC + c
    pltpu.sync_copy(zer1_hbm.at[pl.ds(s * ROWS_S, ROWS_S)],
                    acc_sh.at[pl.ds(s * ROWS_S, ROWS_S)])
    plsc.subcore_barrier()

    def body(i, carry):
        base = wid * CAPW + i * (SGRP * WIN)
        cp0 = pltpu.async_copy(src_hbm.at[pl.ds(base, SGRP * WIN)], sidx, sem_i)
        cps = [pltpu.async_copy(dst_hbm.at[pl.ds(base + j * WIN, WIN)],
                                didx.at[j], sem_i) for j in range(SGRP)]
        cp0.wait()
        gs = [pltpu.async_copy(sn_hbm.at[sidx.at[pl.ds(j * WIN, WIN)]],
                               vals.at[j], sem_g) for j in range(SGRP)]
        ss = []
        for j in range(SGRP):
            cps[j].wait()
            gs[j].wait()
            ss.append(pltpu.async_copy(vals.at[j], acc_sh.at[didx.at[j]],
                                       sem_s, add=True))
        for cp in ss:
            cp.wait()
        return carry

    lax.fori_loop(0, NWIN_T // SGRP, body, 0)
    plsc.subcore_barrier()
    pltpu.sync_copy(acc_sh.at[pl.ds(s * ROWS_S, ROWS_S)],
                    out_hbm.at[c, pl.ds(s * ROWS_S, ROWS_S)])


_ssc_call = pl.kernel(
    _ssc_body,
    out_type=jax.ShapeDtypeStruct((NC, NROWS), jnp.float32),
    mesh=_MESH,
    scratch_types=[
        pltpu.VMEM((SGRP * WIN,), jnp.int32),
        pltpu.VMEM((SGRP, WIN), jnp.int32),
        pltpu.VMEM((SGRP, WIN), jnp.float32),
        pltpu.VMEM_SHARED((NROWS,), jnp.float32),
        pltpu.SemaphoreType.DMA,
        pltpu.SemaphoreType.DMA,
        pltpu.SemaphoreType.DMA,
    ],
)


EGRP = 4


def _edg_body(kept_hbm, src_hbm, dst_hbm, srco_hbm, dsto_hbm,
              sbuf, dbuf, ksv, kdv, sem_i, sem_g):
    c = lax.axis_index("c")
    s = lax.axis_index("s")
    wid = s * NC + c
    lane = lax.iota(jnp.int32, 16)

    def body(i, carry):
        base = wid * CAPW + i * (EGRP * WIN)
        cp0 = pltpu.async_copy(src_hbm.at[pl.ds(base, EGRP * WIN)], sbuf, sem_i)
        cp1 = pltpu.async_copy(dst_hbm.at[pl.ds(base, EGRP * WIN)], dbuf, sem_i)
        cp0.wait()
        cp1.wait()
        gs = [pltpu.async_copy(kept_hbm.at[sbuf.at[pl.ds(j * WIN, WIN)]],
                               ksv.at[j], sem_g) for j in range(EGRP)]
        gd = [pltpu.async_copy(kept_hbm.at[dbuf.at[pl.ds(j * WIN, WIN)]],
                               kdv.at[j], sem_g) for j in range(EGRP)]
        for cp in gs + gd:
            cp.wait()

        for j in range(EGRP):
            def grp_body(q, carry2, j=j):
                off = j * WIN + q * 16
                s_v = sbuf[pl.ds(off, 16)]
                d_v = dbuf[pl.ds(off, 16)]
                ks = ksv[j, pl.ds(q * 16, 16)]
                kd = kdv[j, pl.ds(q * 16, 16)]
                live = (ks * kd) > 0.5
                spread = wid * 577 + i * 131 + off + lane
                sbuf[pl.ds(off, 16)] = jnp.where(live, s_v, spread % NND)
                dbuf[pl.ds(off, 16)] = jnp.where(live, d_v,
                                                 NND + (spread % NTRASH))
                return carry2

            lax.fori_loop(0, WIN // 16, grp_body, 0)
        cp2 = pltpu.async_copy(sbuf, srco_hbm.at[pl.ds(base, EGRP * WIN)], sem_i)
        cp3 = pltpu.async_copy(dbuf, dsto_hbm.at[pl.ds(base, EGRP * WIN)], sem_i)
        cp2.wait()
        cp3.wait()
        return carry

    lax.fori_loop(0, NWIN_T // EGRP, body, 0)


_edg_call = pl.kernel(
    _edg_body,
    out_type=[
        jax.ShapeDtypeStruct((NTE,), jnp.int32),
        jax.ShapeDtypeStruct((NTE,), jnp.int32),
    ],
    mesh=_MESH,
    scratch_types=[
        pltpu.VMEM((EGRP * WIN,), jnp.int32),
        pltpu.VMEM((EGRP * WIN,), jnp.int32),
        pltpu.VMEM((EGRP, WIN), jnp.float32),
        pltpu.VMEM((EGRP, WIN), jnp.float32),
        pltpu.SemaphoreType.DMA,
        pltpu.SemaphoreType.DMA,
    ],
)


def _prep_body(W1, Wf1, b1, bf1, W2, Wf2, b2, bf2, W3, Wf3, b3, bf3,
               Wc1, bc1, Wc2, bc2, Wc3, bc3):
    hi = jax.lax.Precision.HIGHEST
    Wc1[...] = jnp.dot(W1[...], Wf1[...], precision=hi)
    bc1[...] = jnp.dot(b1[...], Wf1[...], precision=hi) + bf1[...]
    Wc2[...] = jnp.dot(W2[...], Wf2[...], precision=hi)
    bc2[...] = jnp.dot(b2[...], Wf2[...], precision=hi) + bf2[...]
    Wc3[...] = jnp.dot(W3[...], Wf3[...], precision=hi)
    bc3[...] = jnp.dot(b3[...], Wf3[...], precision=hi) + bf3[...]


def _prep_call(W1, Wf1, b1, bf1, W2, Wf2, b2, bf2, W3, Wf3, b3, bf3):
    DIN = W1.shape[0]
    return pl.pallas_call(
        _prep_body,
        out_shape=[
            jax.ShapeDtypeStruct((DIN, H), jnp.float32),
            jax.ShapeDtypeStruct((1, H), jnp.float32),
            jax.ShapeDtypeStruct((H, H), jnp.float32),
            jax.ShapeDtypeStruct((1, H), jnp.float32),
            jax.ShapeDtypeStruct((H, H), jnp.float32),
            jax.ShapeDtypeStruct((1, H), jnp.float32),
        ],
    )(W1, Wf1, b1, bf1, W2, Wf2, b2, bf2, W3, Wf3, b3, bf3)


NBLK = 8
RBLK = NROWS // NBLK


def _lin_body(x, Wc, degp, hp, dinv):
    d = degp[0] + degp[1] + 1.0
    di = lax.rsqrt(d)
    h = jnp.dot(x[...], Wc[...], precision=jax.lax.Precision.HIGHEST)
    hp[...] = h * di
    dinv[...] = di


def _lin_call(x, Wc, degp):
    DIN = x.shape[1]
    return pl.pallas_call(
        _lin_body,
        grid=(NBLK,),
        in_specs=[
            pl.BlockSpec((RBLK, DIN), lambda r: (r, 0)),
            pl.BlockSpec((DIN, H), lambda r: (0, 0)),
            pl.BlockSpec((NC, RBLK, 1), lambda r: (0, r, 0)),
        ],
        out_specs=[
            pl.BlockSpec((RBLK, H), lambda r: (r, 0)),
            pl.BlockSpec((RBLK, 1), lambda r: (r, 0)),
        ],
        out_shape=[
            jax.ShapeDtypeStruct((NROWS, H), jnp.float32),
            jax.ShapeDtypeStruct((NROWS, 1), jnp.float32),
        ],
    )(x, Wc, degp)


def _post_body(aggp, hp, dinv, bc, g, be, Wsr, xl, sn, rn):
    y = dinv[...] * (aggp[0] + aggp[1] + hp[...]) + bc[...]
    xv = jnp.maximum(y, 0.0) * (g[...] * GSCALE) + be[...]
    xl[...] = xv
    sr = jnp.dot(xv, Wsr[...], precision=jax.lax.Precision.HIGHEST)
    sn[...] = sr[:, 0:1]
    rn[...] = sr[:, 1:2]


def _post_call(aggp, hp, dinv, bc, g, be, Wsr):
    return pl.pallas_call(
        _post_body,
        grid=(NBLK,),
        in_specs=[
            pl.BlockSpec((NC, RBLK, H), lambda r: (0, r, 0)),
            pl.BlockSpec((RBLK, H), lambda r: (r, 0)),
            pl.BlockSpec((RBLK, 1), lambda r: (r, 0)),
            pl.BlockSpec((1, H), lambda r: (0, 0)),
            pl.BlockSpec((1, H), lambda r: (0, 0)),
            pl.BlockSpec((1, H), lambda r: (0, 0)),
            pl.BlockSpec((H, 2), lambda r: (0, 0)),
        ],
        out_specs=[
            pl.BlockSpec((RBLK, H), lambda r: (r, 0)),
            pl.BlockSpec((RBLK, 1), lambda r: (r, 0)),
            pl.BlockSpec((RBLK, 1), lambda r: (r, 0)),
        ],
        out_shape=[
            jax.ShapeDtypeStruct((NROWS, H), jnp.float32),
            jax.ShapeDtypeStruct((NROWS, 1), jnp.float32),
            jax.ShapeDtypeStruct((NROWS, 1), jnp.float32),
        ],
    )(aggp, hp, dinv, bc, g, be, Wsr)


def _pool_body(k, final, saggp, rn, bs, xl, alive, *outs):
    sc = saggp[0] + saggp[1] + bs[0, 0] + rn[...]
    u = lax.bitcast_convert_type(sc, jnp.int32)
    key = jnp.where(u < 0, (~u) ^ MINKEY, u)
    key = jnp.where(alive[...] > 0.5, key, MINKEY)

    def kth_body(i, lohi):
        lo, hi = lohi
        mid = (lo >> 1) + (hi >> 1) + (lo & hi & 1)
        cnt = jnp.sum((key >= mid).astype(jnp.int32))
        good = cnt >= k
        return jnp.where(good, mid, lo), jnp.where(good, hi, mid)

    t, _ = lax.fori_loop(0, 33, kth_body,
                         (jnp.full((), MINKEY), jnp.full((), np.int32(2**31 - 1))))
    cnt_gt = jnp.sum((key > t).astype(jnp.int32))
    tneed = k - cnt_gt
    idx = lax.broadcasted_iota(jnp.int32, key.shape, 0)
    ties = key == t

    def tie_body(i, lohi):
        lo, hi = lohi
        mid = (lo + hi) // 2
        cnt = jnp.sum((ties & (idx <= mid)).astype(jnp.int32))
        good = cnt >= tneed
        return jnp.where(good, lo, mid), jnp.where(good, mid, hi)

    _, m = lax.fori_loop(0, 15, tie_body,
                         (jnp.full((), np.int32(-1)), jnp.full((), np.int32(NROWS - 1))))
    kept = ((key > t) | (ties & (idx <= m))).astype(jnp.float32)
    tsc = jnp.tanh(sc) * kept
    xp = xl[...] * tsc
    if final:
        outs[0][...] = jnp.sum(xp, axis=0, keepdims=True) * (1.0 / float(k))
    else:
        outs[0][...] = xp
        outs[1][...] = kept


def _pool_call(k, final, saggp, rn, bs, xl, alive):
    if final:
        out_shape = [jax.ShapeDtypeStruct((1, H), jnp.float32)]
    else:
        out_shape = [jax.ShapeDtypeStruct((NROWS, H), jnp.float32),
                     jax.ShapeDtypeStruct((NROWS, 1), jnp.float32)]
    return pl.pallas_call(
        functools.partial(_pool_body, k, final),
        out_shape=out_shape,
    )(saggp, rn, bs, xl, alive)


def kernel(x, edge_index, batch,
           W1, b1, Wf1, bf1, g1, be1, Ws1, bs1, Wr1,
           W2, b2, Wf2, bf2, g2, be2, Ws2, bs2, Wr2,
           W3, b3, Wf3, bf3, g3, be3, Ws3, bs3, Wr3):
    f32 = jnp.float32
    x = x.astype(f32)
    src = edge_index[0].astype(jnp.int32).reshape(NT, E // NT)
    dst = edge_index[1].astype(jnp.int32).reshape(NT, E // NT)
    npad = CAPW - E // NT
    t_i = jnp.arange(NT, dtype=jnp.int32)[:, None]
    j_i = jnp.arange(npad, dtype=jnp.int32)[None, :]
    pad_s = (t_i * 313 + j_i) % NND
    pad_d = NND + (t_i * 37 + j_i) % NTRASH
    srcC = jnp.concatenate([src, pad_s], axis=1).reshape(-1)
    dstC = jnp.concatenate([dst, pad_d], axis=1).reshape(-1)

    xp = jnp.zeros((NROWS, x.shape[1]), f32).at[:NND].set(x)
    alive = (jnp.arange(NROWS) < NND).astype(f32).reshape(NROWS, 1)
    zer1 = jnp.zeros((NROWS,), f32)
    zer2 = jnp.zeros((NROWS, H), f32)

    Wc1, bc1, Wc2, bc2, Wc3, bc3 = _prep_call(
        W1, Wf1, b1.reshape(1, H), bf1.reshape(1, H),
        W2, Wf2, b2.reshape(1, H), bf2.reshape(1, H),
        W3, Wf3, b3.reshape(1, H), bf3.reshape(1, H))

    layer_params = [
        (Wc1, bc1, g1, be1, Ws1, bs1, Wr1, 5000),
        (Wc2, bc2, g2, be2, Ws2, bs2, Wr2, 2500),
        (Wc3, bc3, g3, be3, Ws3, bs3, Wr3, 1250),
    ]

    for li, (Wc, bc, g, be, Ws, bs, Wr, k) in enumerate(layer_params):
        final = li == 2
        degp = _deg_call(dstC, zer1)
        hp, dinv = _lin_call(xp, Wc, degp.reshape(NC, NROWS, 1))
        aggp = _agg_call(hp, srcC, dstC, zer2)
        xl, sn, rn = _post_call(aggp, hp, dinv, bc, g.reshape(1, H),
                                be.reshape(1, H),
                                jnp.concatenate([Ws, Wr], axis=1))
        saggp = _ssc_call(sn.reshape(NROWS), srcC, dstC, zer1)
        outs = _pool_call(k, final, saggp.reshape(NC, NROWS, 1), rn,
                          bs.reshape(1, 1), xl, alive)
        if final:
            return outs[0]
        xp, kept = outs
        srcC, dstC = _edg_call(kept.reshape(NROWS), srcC, dstC)
        alive = kept


# speedup vs baseline: 56.9476x; 1.4824x over previous
import functools
import numpy as np
import jax
import jax.numpy as jnp
from jax import lax
from jax.experimental import pallas as pl
from jax.experimental.pallas import tpu as pltpu
from jax.experimental.pallas import tpu_sc as plsc

NND = 10000
NROWS = 10240
NTRASH = NROWS - NND
E = 640000
H = 128
NC, NS = 2, 16
NT = NC * NS
WIN = 128
CAPW = 20480
NWIN_T = CAPW // WIN
NTE = NT * CAPW
ROWS_S = NROWS // NS
GSCALE = float(1.0 / np.sqrt(1.0 + 1e-5))
MINKEY = np.int32(-2**31)

_MESH = plsc.VectorSubcoreMesh(core_axis_name="c", subcore_axis_name="s")


DGRP = 4


def _deg_body(dst_hbm, zer1_hbm, degp_hbm, didx, ones_v, acc_sh, sem_i, sem_s):
    c = lax.axis_index("c")
    s = lax.axis_index("s")
    wid = s * NC + c
    for i in range(WIN // 16):
        ones_v[pl.ds(i * 16, 16)] = jnp.ones((16,), jnp.float32)
    pltpu.sync_copy(zer1_hbm.at[pl.ds(s * ROWS_S, ROWS_S)],
                    acc_sh.at[pl.ds(s * ROWS_S, ROWS_S)])
    plsc.subcore_barrier()

    def body(i, carry):
        base = wid * CAPW + i * (DGRP * WIN)
        cps = [pltpu.async_copy(dst_hbm.at[pl.ds(base + j * WIN, WIN)],
                                didx.at[j], sem_i) for j in range(DGRP)]
        sc_ = []
        for j in range(DGRP):
            cps[j].wait()
            sc_.append(pltpu.async_copy(ones_v, acc_sh.at[didx.at[j]],
                                        sem_s, add=True))
        for cp in sc_:
            cp.wait()
        return carry

    lax.fori_loop(0, NWIN_T // DGRP, body, 0)
    plsc.subcore_barrier()
    pltpu.sync_copy(acc_sh.at[pl.ds(s * ROWS_S, ROWS_S)],
                    degp_hbm.at[c, pl.ds(s * ROWS_S, ROWS_S)])


_deg_call = pl.kernel(
    _deg_body,
    out_type=jax.ShapeDtypeStruct((NC, NROWS), jnp.float32),
    mesh=_MESH,
    scratch_types=[
        pltpu.VMEM((DGRP, WIN), jnp.int32),
        pltpu.VMEM((WIN,), jnp.float32),
        pltpu.VMEM_SHARED((NROWS,), jnp.float32),
        pltpu.SemaphoreType.DMA,
        pltpu.SemaphoreType.DMA,
    ],
)


AGRP = 2


def _agg_body(hp_hbm, src_hbm, dst_hbm, zer2_hbm, out_hbm,
              sidx, didx, rows, acc_sh, sem_i, sem_g, sem_s):
    c = lax.axis_index("c")
    s = lax.axis_index("s")
    wid = s * NC + c
    pltpu.sync_copy(zer2_hbm.at[pl.ds(s * ROWS_S, ROWS_S)],
                    acc_sh.at[pl.ds(s * ROWS_S, ROWS_S)])
    plsc.subcore_barrier()

    def body(i, carry):
        base = wid * CAPW + i * (AGRP * WIN)
        cp0 = pltpu.async_copy(src_hbm.at[pl.ds(base, AGRP * WIN)], sidx, sem_i)
        cps = [pltpu.async_copy(dst_hbm.at[pl.ds(base + j * WIN, WIN)],
                                didx.at[j], sem_i) for j in range(AGRP)]
        cp0.wait()
        gs = [pltpu.async_copy(hp_hbm.at[sidx.at[pl.ds(j * WIN, WIN)]],
                               rows.at[j], sem_g) for j in range(AGRP)]
        ss = []
        for j in range(AGRP):
            cps[j].wait()
            gs[j].wait()
            ss.append(pltpu.async_copy(rows.at[j], acc_sh.at[didx.at[j]],
                                       sem_s, add=True))
        for cp in ss:
            cp.wait()
        return carry

    lax.fori_loop(0, NWIN_T // AGRP, body, 0)
    plsc.subcore_barrier()
    pltpu.sync_copy(acc_sh.at[pl.ds(s * ROWS_S, ROWS_S)],
                    out_hbm.at[c, pl.ds(s * ROWS_S, ROWS_S)])


_agg_call = pl.kernel(
    _agg_body,
    out_type=jax.ShapeDtypeStruct((NC, NROWS, H), jnp.float32),
    mesh=_MESH,
    scratch_types=[
        pltpu.VMEM((AGRP * WIN,), jnp.int32),
        pltpu.VMEM((AGRP, WIN), jnp.int32),
        pltpu.VMEM((AGRP, WIN, H), jnp.float32),
        pltpu.VMEM_SHARED((NROWS, H), jnp.float32),
        pltpu.SemaphoreType.DMA,
        pltpu.SemaphoreType.DMA,
        pltpu.SemaphoreType.DMA,
    ],
)


SGRP = 4


def _ssc_body(sn_hbm, src_hbm, dst_hbm, zer1_hbm, out_hbm,
              sidx, didx, vals, acc_sh, sem_i, sem_g, sem_s):
    c = lax.axis_index("c")
    s = lax.axis_index("s")
    wid = s * NC + c
    pltpu.sync_copy(zer1_hbm.at[pl.ds(s * ROWS_S, ROWS_S)],
                    acc_sh.at[pl.ds(s * ROWS_S, ROWS_S)])
    plsc.subcore_barrier()

    def body(i, carry):
        base = wid * CAPW + i * (SGRP * WIN)
        cp0 = pltpu.async_copy(src_hbm.at[pl.ds(base, SGRP * WIN)], sidx, sem_i)
        cps = [pltpu.async_copy(dst_hbm.at[pl.ds(base + j * WIN, WIN)],
                                didx.at[j], sem_i) for j in range(SGRP)]
        cp0.wait()
        gs = [pltpu.async_copy(sn_hbm.at[sidx.at[pl.ds(j * WIN, WIN)]],
                               vals.at[j], sem_g) for j in range(SGRP)]
        ss = []
        for j in range(SGRP):
            cps[j].wait()
            gs[j].wait()
            ss.append(pltpu.async_copy(vals.at[j], acc_sh.at[didx.at[j]],
                                       sem_s, add=True))
        for cp in ss:
            cp.wait()
        return carry

    lax.fori_loop(0, NWIN_T // SGRP, body, 0)
    plsc.subcore_barrier()
    pltpu.sync_copy(acc_sh.at[pl.ds(s * ROWS_S, ROWS_S)],
                    out_hbm.at[c, pl.ds(s * ROWS_S, ROWS_S)])


_ssc_call = pl.kernel(
    _ssc_body,
    out_type=jax.ShapeDtypeStruct((NC, NROWS), jnp.float32),
    mesh=_MESH,
    scratch_types=[
        pltpu.VMEM((SGRP * WIN,), jnp.int32),
        pltpu.VMEM((SGRP, WIN), jnp.int32),
        pltpu.VMEM((SGRP, WIN), jnp.float32),
        pltpu.VMEM_SHARED((NROWS,), jnp.float32),
        pltpu.SemaphoreType.DMA,
        pltpu.SemaphoreType.DMA,
        pltpu.SemaphoreType.DMA,
    ],
)


EGRP = 4


def _edg_body(kept_hbm, src_hbm, dst_hbm, srco_hbm, dsto_hbm,
              sbuf, dbuf, dgid, ksv, kdv, sem_i, sem_g):
    c = lax.axis_index("c")
    s = lax.axis_index("s")
    wid = s * NC + c
    lane = lax.iota(jnp.int32, 16)

    def body(i, carry):
        base = wid * CAPW + i * (EGRP * WIN)
        cp0 = pltpu.async_copy(src_hbm.at[pl.ds(base, EGRP * WIN)], sbuf, sem_i)
        cp1 = pltpu.async_copy(dst_hbm.at[pl.ds(base, EGRP * WIN)], dbuf, sem_i)
        cp0.wait()
        gs = [pltpu.async_copy(kept_hbm.at[sbuf.at[pl.ds(j * WIN, WIN)]],
                               ksv.at[j], sem_g) for j in range(EGRP)]
        cp1.wait()

        def pre_body(q, carry2):
            off = q * 16
            d_v = dbuf[pl.ds(off, 16)]
            spread = wid * 577 + i * 131 + off + lane
            dgid[pl.ds(off, 16)] = jnp.where(d_v < NND, d_v, spread % NND)
            return carry2

        lax.fori_loop(0, (EGRP * WIN) // 16, pre_body, 0)
        gd = [pltpu.async_copy(kept_hbm.at[dgid.at[pl.ds(j * WIN, WIN)]],
                               kdv.at[j], sem_g) for j in range(EGRP)]
        for cp in gs + gd:
            cp.wait()

        for j in range(EGRP):
            def grp_body(q, carry2, j=j):
                off = j * WIN + q * 16
                s_v = sbuf[pl.ds(off, 16)]
                d_v = dbuf[pl.ds(off, 16)]
                ks = ksv[j, pl.ds(q * 16, 16)]
                kd = kdv[j, pl.ds(q * 16, 16)]
                live = ((ks * kd) > 0.5) & (d_v < NND)
                spread = wid * 577 + i * 131 + off + lane
                sbuf[pl.ds(off, 16)] = jnp.where(live, s_v, spread % NND)
                dbuf[pl.ds(off, 16)] = jnp.where(live, d_v,
                                                 NND + (spread % NTRASH))
                return carry2

            lax.fori_loop(0, WIN // 16, grp_body, 0)
        cp2 = pltpu.async_copy(sbuf, srco_hbm.at[pl.ds(base, EGRP * WIN)], sem_i)
        cp3 = pltpu.async_copy(dbuf, dsto_hbm.at[pl.ds(base, EGRP * WIN)], sem_i)
        cp2.wait()
        cp3.wait()
        return carry

    lax.fori_loop(0, NWIN_T // EGRP, body, 0)


_edg_call = pl.kernel(
    _edg_body,
    out_type=[
        jax.ShapeDtypeStruct((NTE,), jnp.int32),
        jax.ShapeDtypeStruct((NTE,), jnp.int32),
    ],
    mesh=_MESH,
    scratch_types=[
        pltpu.VMEM((EGRP * WIN,), jnp.int32),
        pltpu.VMEM((EGRP * WIN,), jnp.int32),
        pltpu.VMEM((EGRP * WIN,), jnp.int32),
        pltpu.VMEM((EGRP, WIN), jnp.float32),
        pltpu.VMEM((EGRP, WIN), jnp.float32),
        pltpu.SemaphoreType.DMA,
        pltpu.SemaphoreType.DMA,
    ],
)


def _prep_body(W1, Wf1, b1, bf1, W2, Wf2, b2, bf2, W3, Wf3, b3, bf3,
               Wc1, bc1, Wc2, bc2, Wc3, bc3):
    hi = jax.lax.Precision.HIGHEST
    Wc1[...] = jnp.dot(W1[...], Wf1[...], precision=hi)
    bc1[...] = jnp.dot(b1[...], Wf1[...], precision=hi) + bf1[...]
    Wc2[...] = jnp.dot(W2[...], Wf2[...], precision=hi)
    bc2[...] = jnp.dot(b2[...], Wf2[...], precision=hi) + bf2[...]
    Wc3[...] = jnp.dot(W3[...], Wf3[...], precision=hi)
    bc3[...] = jnp.dot(b3[...], Wf3[...], precision=hi) + bf3[...]


def _prep_call(W1, Wf1, b1, bf1, W2, Wf2, b2, bf2, W3, Wf3, b3, bf3):
    DIN = W1.shape[0]
    return pl.pallas_call(
        _prep_body,
        out_shape=[
            jax.ShapeDtypeStruct((DIN, H), jnp.float32),
            jax.ShapeDtypeStruct((1, H), jnp.float32),
            jax.ShapeDtypeStruct((H, H), jnp.float32),
            jax.ShapeDtypeStruct((1, H), jnp.float32),
            jax.ShapeDtypeStruct((H, H), jnp.float32),
            jax.ShapeDtypeStruct((1, H), jnp.float32),
        ],
    )(W1, Wf1, b1, bf1, W2, Wf2, b2, bf2, W3, Wf3, b3, bf3)


NBLK = 8
RBLK = NROWS // NBLK


def _lin_body(x, Wc, degp, hp, dinv):
    d = degp[0] + degp[1] + 1.0
    di = lax.rsqrt(d)
    h = jnp.dot(x[...], Wc[...], precision=jax.lax.Precision.HIGHEST)
    hp[...] = h * di
    dinv[...] = di


def _lin_call(x, Wc, degp):
    DIN = x.shape[1]
    return pl.pallas_call(
        _lin_body,
        grid=(NBLK,),
        in_specs=[
            pl.BlockSpec((RBLK, DIN), lambda r: (r, 0)),
            pl.BlockSpec((DIN, H), lambda r: (0, 0)),
            pl.BlockSpec((NC, RBLK, 1), lambda r: (0, r, 0)),
        ],
        out_specs=[
            pl.BlockSpec((RBLK, H), lambda r: (r, 0)),
            pl.BlockSpec((RBLK, 1), lambda r: (r, 0)),
        ],
        out_shape=[
            jax.ShapeDtypeStruct((NROWS, H), jnp.float32),
            jax.ShapeDtypeStruct((NROWS, 1), jnp.float32),
        ],
    )(x, Wc, degp)


def _post_body(aggp, hp, dinv, bc, g, be, Wsr, xl, sn, rn):
    y = dinv[...] * (aggp[0] + aggp[1] + hp[...]) + bc[...]
    xv = jnp.maximum(y, 0.0) * (g[...] * GSCALE) + be[...]
    xl[...] = xv
    sr = jnp.dot(xv, Wsr[...], precision=jax.lax.Precision.HIGHEST)
    sn[...] = sr[:, 0:1]
    rn[...] = sr[:, 1:2]


def _post_call(aggp, hp, dinv, bc, g, be, Wsr):
    return pl.pallas_call(
        _post_body,
        grid=(NBLK,),
        in_specs=[
            pl.BlockSpec((NC, RBLK, H), lambda r: (0, r, 0)),
            pl.BlockSpec((RBLK, H), lambda r: (r, 0)),
            pl.BlockSpec((RBLK, 1), lambda r: (r, 0)),
            pl.BlockSpec((1, H), lambda r: (0, 0)),
            pl.BlockSpec((1, H), lambda r: (0, 0)),
            pl.BlockSpec((1, H), lambda r: (0, 0)),
            pl.BlockSpec((H, 2), lambda r: (0, 0)),
        ],
        out_specs=[
            pl.BlockSpec((RBLK, H), lambda r: (r, 0)),
            pl.BlockSpec((RBLK, 1), lambda r: (r, 0)),
            pl.BlockSpec((RBLK, 1), lambda r: (r, 0)),
        ],
        out_shape=[
            jax.ShapeDtypeStruct((NROWS, H), jnp.float32),
            jax.ShapeDtypeStruct((NROWS, 1), jnp.float32),
            jax.ShapeDtypeStruct((NROWS, 1), jnp.float32),
        ],
    )(aggp, hp, dinv, bc, g, be, Wsr)


def _pool_body(k, final, saggp, rn, bs, xl, alive, *outs):
    sc = saggp[0] + saggp[1] + bs[0, 0] + rn[...]
    u = lax.bitcast_convert_type(sc, jnp.int32)
    key = jnp.where(u < 0, (~u) ^ MINKEY, u)
    key = jnp.where(alive[...] > 0.5, key, MINKEY)

    def kth_body(i, lohi):
        lo, hi = lohi
        mid = (lo >> 1) + (hi >> 1) + (lo & hi & 1)
        cnt = jnp.sum((key >= mid).astype(jnp.int32))
        good = cnt >= k
        return jnp.where(good, mid, lo), jnp.where(good, hi, mid)

    t, _ = lax.fori_loop(0, 33, kth_body,
                         (jnp.full((), MINKEY), jnp.full((), np.int32(2**31 - 1))))
    cnt_gt = jnp.sum((key > t).astype(jnp.int32))
    tneed = k - cnt_gt
    idx = lax.broadcasted_iota(jnp.int32, key.shape, 0)
    ties = key == t

    def tie_body(i, lohi):
        lo, hi = lohi
        mid = (lo + hi) // 2
        cnt = jnp.sum((ties & (idx <= mid)).astype(jnp.int32))
        good = cnt >= tneed
        return jnp.where(good, lo, mid), jnp.where(good, mid, hi)

    _, m = lax.fori_loop(0, 15, tie_body,
                         (jnp.full((), np.int32(-1)), jnp.full((), np.int32(NROWS - 1))))
    kept = ((key > t) | (ties & (idx <= m))).astype(jnp.float32)
    tsc = jnp.tanh(sc) * kept
    xp = xl[...] * tsc
    if final:
        outs[0][...] = jnp.sum(xp, axis=0, keepdims=True) * (1.0 / float(k))
    else:
        outs[0][...] = xp
        outs[1][...] = kept


def _pool_call(k, final, saggp, rn, bs, xl, alive):
    if final:
        out_shape = [jax.ShapeDtypeStruct((1, H), jnp.float32)]
    else:
        out_shape = [jax.ShapeDtypeStruct((NROWS, H), jnp.float32),
                     jax.ShapeDtypeStruct((NROWS, 1), jnp.float32)]
    return pl.pallas_call(
        functools.partial(_pool_body, k, final),
        out_shape=out_shape,
    )(saggp, rn, bs, xl, alive)


def kernel(x, edge_index, batch,
           W1, b1, Wf1, bf1, g1, be1, Ws1, bs1, Wr1,
           W2, b2, Wf2, bf2, g2, be2, Ws2, bs2, Wr2,
           W3, b3, Wf3, bf3, g3, be3, Ws3, bs3, Wr3):
    f32 = jnp.float32
    x = x.astype(f32)
    src = edge_index[0].astype(jnp.int32).reshape(NT, E // NT)
    dst = edge_index[1].astype(jnp.int32).reshape(NT, E // NT)
    npad = CAPW - E // NT
    t_i = jnp.arange(NT, dtype=jnp.int32)[:, None]
    j_i = jnp.arange(npad, dtype=jnp.int32)[None, :]
    pad_s = (t_i * 313 + j_i) % NND
    pad_d = NND + (t_i * 37 + j_i) % NTRASH
    srcC = jnp.concatenate([src, pad_s], axis=1).reshape(-1)
    dstC = jnp.concatenate([dst, pad_d], axis=1).reshape(-1)

    xp = jnp.zeros((NROWS, x.shape[1]), f32).at[:NND].set(x)
    alive = (jnp.arange(NROWS) < NND).astype(f32).reshape(NROWS, 1)
    zer1 = jnp.zeros((NROWS,), f32)
    zer2 = jnp.zeros((NROWS, H), f32)

    Wc1, bc1, Wc2, bc2, Wc3, bc3 = _prep_call(
        W1, Wf1, b1.reshape(1, H), bf1.reshape(1, H),
        W2, Wf2, b2.reshape(1, H), bf2.reshape(1, H),
        W3, Wf3, b3.reshape(1, H), bf3.reshape(1, H))

    layer_params = [
        (Wc1, bc1, g1, be1, Ws1, bs1, Wr1, 5000),
        (Wc2, bc2, g2, be2, Ws2, bs2, Wr2, 2500),
        (Wc3, bc3, g3, be3, Ws3, bs3, Wr3, 1250),
    ]

    for li, (Wc, bc, g, be, Ws, bs, Wr, k) in enumerate(layer_params):
        final = li == 2
        degp = _deg_call(dstC, zer1)
        hp, dinv = _lin_call(xp, Wc, degp.reshape(NC, NROWS, 1))
        aggp = _agg_call(hp, srcC, dstC, zer2)
        xl, sn, rn = _post_call(aggp, hp, dinv, bc, g.reshape(1, H),
                                be.reshape(1, H),
                                jnp.concatenate([Ws, Wr], axis=1))
        saggp = _ssc_call(sn.reshape(NROWS), srcC, dstC, zer1)
        outs = _pool_call(k, final, saggp.reshape(NC, NROWS, 1), rn,
                          bs.reshape(1, 1), xl, alive)
        if final:
            return outs[0]
        xp, kept = outs
        srcC, dstC = _edg_call(kept.reshape(NROWS), srcC, dstC)
        alive = kept


# speedup vs baseline: 59.4860x; 1.0446x over previous
import functools
import numpy as np
import jax
import jax.numpy as jnp
from jax import lax
from jax.experimental import pallas as pl
from jax.experimental.pallas import tpu as pltpu
from jax.experimental.pallas import tpu_sc as plsc

NND = 10000
NROWS = 10240
NTRASH = NROWS - NND
E = 640000
H = 128
NC, NS = 2, 16
NT = NC * NS
WIN = 128
CAPW = 20480
NWIN_T = CAPW // WIN
NTE = NT * CAPW
ROWS_S = NROWS // NS
GSCALE = float(1.0 / np.sqrt(1.0 + 1e-5))
MINKEY = np.int32(-2**31)

_MESH = plsc.VectorSubcoreMesh(core_axis_name="c", subcore_axis_name="s")


DGRP = 4


def _deg_body(dst_hbm, zer1_hbm, degp_hbm, didx, ones_v, acc_sh, sem_i, sem_s):
    c = lax.axis_index("c")
    s = lax.axis_index("s")
    wid = s * NC + c
    for i in range(WIN // 16):
        ones_v[pl.ds(i * 16, 16)] = jnp.ones((16,), jnp.float32)
    pltpu.sync_copy(zer1_hbm.at[pl.ds(s * ROWS_S, ROWS_S)],
                    acc_sh.at[pl.ds(s * ROWS_S, ROWS_S)])
    plsc.subcore_barrier()

    def body(i, carry):
        base = wid * CAPW + i * (DGRP * WIN)
        cps = [pltpu.async_copy(dst_hbm.at[pl.ds(base + j * WIN, WIN)],
                                didx.at[j], sem_i) for j in range(DGRP)]
        sc_ = []
        for j in range(DGRP):
            cps[j].wait()
            sc_.append(pltpu.async_copy(ones_v, acc_sh.at[didx.at[j]],
                                        sem_s, add=True))
        for cp in sc_:
            cp.wait()
        return carry

    lax.fori_loop(0, NWIN_T // DGRP, body, 0)
    plsc.subcore_barrier()
    pltpu.sync_copy(acc_sh.at[pl.ds(s * ROWS_S, ROWS_S)],
                    degp_hbm.at[c, pl.ds(s * ROWS_S, ROWS_S)])


_deg_call = pl.kernel(
    _deg_body,
    out_type=jax.ShapeDtypeStruct((NC, NROWS), jnp.float32),
    mesh=_MESH,
    scratch_types=[
        pltpu.VMEM((DGRP, WIN), jnp.int32),
        pltpu.VMEM((WIN,), jnp.float32),
        pltpu.VMEM_SHARED((NROWS,), jnp.float32),
        pltpu.SemaphoreType.DMA,
        pltpu.SemaphoreType.DMA,
    ],
)


AGRP = 2


def _agg_body(hp_hbm, src_hbm, dst_hbm, zer2_hbm, out_hbm,
              sidx, didx, rows, acc_sh, sem_i, sem_g, sem_s):
    c = lax.axis_index("c")
    s = lax.axis_index("s")
    wid = s * NC + c
    pltpu.sync_copy(zer2_hbm.at[pl.ds(s * ROWS_S, ROWS_S)],
                    acc_sh.at[pl.ds(s * ROWS_S, ROWS_S)])
    plsc.subcore_barrier()

    def body(i, carry):
        base = wid * CAPW + i * (AGRP * WIN)
        cp0 = pltpu.async_copy(src_hbm.at[pl.ds(base, AGRP * WIN)], sidx, sem_i)
        cps = [pltpu.async_copy(dst_hbm.at[pl.ds(base + j * WIN, WIN)],
                                didx.at[j], sem_i) for j in range(AGRP)]
        cp0.wait()
        gs = [pltpu.async_copy(hp_hbm.at[sidx.at[pl.ds(j * WIN, WIN)]],
                               rows.at[j], sem_g) for j in range(AGRP)]
        ss = []
        for j in range(AGRP):
            cps[j].wait()
            gs[j].wait()
            ss.append(pltpu.async_copy(rows.at[j], acc_sh.at[didx.at[j]],
                                       sem_s, add=True))
        for cp in ss:
            cp.wait()
        return carry

    lax.fori_loop(0, NWIN_T // AGRP, body, 0)
    plsc.subcore_barrier()
    pltpu.sync_copy(acc_sh.at[pl.ds(s * ROWS_S, ROWS_S)],
                    out_hbm.at[c, pl.ds(s * ROWS_S, ROWS_S)])


_agg_call = pl.kernel(
    _agg_body,
    out_type=jax.ShapeDtypeStruct((NC, NROWS, H), jnp.float32),
    mesh=_MESH,
    scratch_types=[
        pltpu.VMEM((AGRP * WIN,), jnp.int32),
        pltpu.VMEM((AGRP, WIN), jnp.int32),
        pltpu.VMEM((AGRP, WIN, H), jnp.float32),
        pltpu.VMEM_SHARED((NROWS, H), jnp.float32),
        pltpu.SemaphoreType.DMA,
        pltpu.SemaphoreType.DMA,
        pltpu.SemaphoreType.DMA,
    ],
)


SGRP = 4


def _ssc_body(sn_hbm, src_hbm, dst_hbm, zer1_hbm, out_hbm,
              sidx, didx, vals, acc_sh, sem_i, sem_g, sem_s):
    c = lax.axis_index("c")
    s = lax.axis_index("s")
    wid = s * NC + c
    pltpu.sync_copy(zer1_hbm.at[pl.ds(s * ROWS_S, ROWS_S)],
                    acc_sh.at[pl.ds(s * ROWS_S, ROWS_S)])
    plsc.subcore_barrier()

    def body(i, carry):
        base = wid * CAPW + i * (SGRP * WIN)
        cp0 = pltpu.async_copy(src_hbm.at[pl.ds(base, SGRP * WIN)], sidx, sem_i)
        cps = [pltpu.async_copy(dst_hbm.at[pl.ds(base + j * WIN, WIN)],
                                didx.at[j], sem_i) for j in range(SGRP)]
        cp0.wait()
        gs = [pltpu.async_copy(sn_hbm.at[sidx.at[pl.ds(j * WIN, WIN)]],
                               vals.at[j], sem_g) for j in range(SGRP)]
        ss = []
        for j in range(SGRP):
            cps[j].wait()
            gs[j].wait()
            ss.append(pltpu.async_copy(vals.at[j], acc_sh.at[didx.at[j]],
                                       sem_s, add=True))
        for cp in ss:
            cp.wait()
        return carry

    lax.fori_loop(0, NWIN_T // SGRP, body, 0)
    plsc.subcore_barrier()
    pltpu.sync_copy(acc_sh.at[pl.ds(s * ROWS_S, ROWS_S)],
                    out_hbm.at[c, pl.ds(s * ROWS_S, ROWS_S)])


_ssc_call = pl.kernel(
    _ssc_body,
    out_type=jax.ShapeDtypeStruct((NC, NROWS), jnp.float32),
    mesh=_MESH,
    scratch_types=[
        pltpu.VMEM((SGRP * WIN,), jnp.int32),
        pltpu.VMEM((SGRP, WIN), jnp.int32),
        pltpu.VMEM((SGRP, WIN), jnp.float32),
        pltpu.VMEM_SHARED((NROWS,), jnp.float32),
        pltpu.SemaphoreType.DMA,
        pltpu.SemaphoreType.DMA,
        pltpu.SemaphoreType.DMA,
    ],
)


EGRP = 4


def _edg_body(kept_hbm, src_hbm, dst_hbm, zer1_hbm, srco_hbm, dsto_hbm,
              degp_hbm, sbuf, dbuf, dgid, ksv, kdv, ones_v, acc_sh,
              sem_i, sem_g, sem_s):
    c = lax.axis_index("c")
    s = lax.axis_index("s")
    wid = s * NC + c
    lane = lax.iota(jnp.int32, 16)
    for n in range(WIN // 16):
        ones_v[pl.ds(n * 16, 16)] = jnp.ones((16,), jnp.float32)
    pltpu.sync_copy(zer1_hbm.at[pl.ds(s * ROWS_S, ROWS_S)],
                    acc_sh.at[pl.ds(s * ROWS_S, ROWS_S)])
    plsc.subcore_barrier()

    def body(i, carry):
        base = wid * CAPW + i * (EGRP * WIN)
        cp0 = pltpu.async_copy(src_hbm.at[pl.ds(base, EGRP * WIN)], sbuf, sem_i)
        cp1 = pltpu.async_copy(dst_hbm.at[pl.ds(base, EGRP * WIN)], dbuf, sem_i)
        cp0.wait()
        gs = [pltpu.async_copy(kept_hbm.at[sbuf.at[pl.ds(j * WIN, WIN)]],
                               ksv.at[j], sem_g) for j in range(EGRP)]
        cp1.wait()

        def pre_body(q, carry2):
            off = q * 16
            d_v = dbuf[pl.ds(off, 16)]
            spread = wid * 577 + i * 131 + off + lane
            dgid[pl.ds(off, 16)] = jnp.where(d_v < NND, d_v, spread % NND)
            return carry2

        lax.fori_loop(0, (EGRP * WIN) // 16, pre_body, 0)
        gd = [pltpu.async_copy(kept_hbm.at[dgid.at[pl.ds(j * WIN, WIN)]],
                               kdv.at[j], sem_g) for j in range(EGRP)]
        for cp in gs + gd:
            cp.wait()

        for j in range(EGRP):
            def grp_body(q, carry2, j=j):
                off = j * WIN + q * 16
                s_v = sbuf[pl.ds(off, 16)]
                d_v = dbuf[pl.ds(off, 16)]
                ks = ksv[j, pl.ds(q * 16, 16)]
                kd = kdv[j, pl.ds(q * 16, 16)]
                live = ((ks * kd) > 0.5) & (d_v < NND)
                spread = wid * 577 + i * 131 + off + lane
                sbuf[pl.ds(off, 16)] = jnp.where(live, s_v, spread % NND)
                dbuf[pl.ds(off, 16)] = jnp.where(live, d_v,
                                                 NND + (spread % NTRASH))
                return carry2

            lax.fori_loop(0, WIN // 16, grp_body, 0)
        cp2 = pltpu.async_copy(sbuf, srco_hbm.at[pl.ds(base, EGRP * WIN)], sem_i)
        cp3 = pltpu.async_copy(dbuf, dsto_hbm.at[pl.ds(base, EGRP * WIN)], sem_i)
        ds_ = [pltpu.async_copy(ones_v, acc_sh.at[dbuf.at[pl.ds(j * WIN, WIN)]],
                                sem_s, add=True) for j in range(EGRP)]
        cp2.wait()
        cp3.wait()
        for cp in ds_:
            cp.wait()
        return carry

    lax.fori_loop(0, NWIN_T // EGRP, body, 0)
    plsc.subcore_barrier()
    pltpu.sync_copy(acc_sh.at[pl.ds(s * ROWS_S, ROWS_S)],
                    degp_hbm.at[c, pl.ds(s * ROWS_S, ROWS_S)])


_edg_call = pl.kernel(
    _edg_body,
    out_type=[
        jax.ShapeDtypeStruct((NTE,), jnp.int32),
        jax.ShapeDtypeStruct((NTE,), jnp.int32),
        jax.ShapeDtypeStruct((NC, NROWS), jnp.float32),
    ],
    mesh=_MESH,
    scratch_types=[
        pltpu.VMEM((EGRP * WIN,), jnp.int32),
        pltpu.VMEM((EGRP * WIN,), jnp.int32),
        pltpu.VMEM((EGRP * WIN,), jnp.int32),
        pltpu.VMEM((EGRP, WIN), jnp.float32),
        pltpu.VMEM((EGRP, WIN), jnp.float32),
        pltpu.VMEM((WIN,), jnp.float32),
        pltpu.VMEM_SHARED((NROWS,), jnp.float32),
        pltpu.SemaphoreType.DMA,
        pltpu.SemaphoreType.DMA,
        pltpu.SemaphoreType.DMA,
    ],
)


def _prep_body(W1, Wf1, b1, bf1, W2, Wf2, b2, bf2, W3, Wf3, b3, bf3,
               Wc1, bc1, Wc2, bc2, Wc3, bc3):
    hi = jax.lax.Precision.HIGHEST
    Wc1[...] = jnp.dot(W1[...], Wf1[...], precision=hi)
    bc1[...] = jnp.dot(b1[...], Wf1[...], precision=hi) + bf1[...]
    Wc2[...] = jnp.dot(W2[...], Wf2[...], precision=hi)
    bc2[...] = jnp.dot(b2[...], Wf2[...], precision=hi) + bf2[...]
    Wc3[...] = jnp.dot(W3[...], Wf3[...], precision=hi)
    bc3[...] = jnp.dot(b3[...], Wf3[...], precision=hi) + bf3[...]


def _prep_call(W1, Wf1, b1, bf1, W2, Wf2, b2, bf2, W3, Wf3, b3, bf3):
    DIN = W1.shape[0]
    return pl.pallas_call(
        _prep_body,
        out_shape=[
            jax.ShapeDtypeStruct((DIN, H), jnp.float32),
            jax.ShapeDtypeStruct((1, H), jnp.float32),
            jax.ShapeDtypeStruct((H, H), jnp.float32),
            jax.ShapeDtypeStruct((1, H), jnp.float32),
            jax.ShapeDtypeStruct((H, H), jnp.float32),
            jax.ShapeDtypeStruct((1, H), jnp.float32),
        ],
    )(W1, Wf1, b1, bf1, W2, Wf2, b2, bf2, W3, Wf3, b3, bf3)


NBLK = 8
RBLK = NROWS // NBLK


def _lin_body(x, Wc, degp, hp, dinv):
    d = degp[0] + degp[1] + 1.0
    di = lax.rsqrt(d)
    h = jnp.dot(x[...], Wc[...], precision=jax.lax.Precision.HIGHEST)
    hp[...] = h * di
    dinv[...] = di


def _lin_call(x, Wc, degp):
    DIN = x.shape[1]
    return pl.pallas_call(
        _lin_body,
        grid=(NBLK,),
        in_specs=[
            pl.BlockSpec((RBLK, DIN), lambda r: (r, 0)),
            pl.BlockSpec((DIN, H), lambda r: (0, 0)),
            pl.BlockSpec((NC, RBLK, 1), lambda r: (0, r, 0)),
        ],
        out_specs=[
            pl.BlockSpec((RBLK, H), lambda r: (r, 0)),
            pl.BlockSpec((RBLK, 1), lambda r: (r, 0)),
        ],
        out_shape=[
            jax.ShapeDtypeStruct((NROWS, H), jnp.float32),
            jax.ShapeDtypeStruct((NROWS, 1), jnp.float32),
        ],
    )(x, Wc, degp)


def _post_body(aggp, hp, dinv, bc, g, be, Wsr, xl, sn, rn):
    y = dinv[...] * (aggp[0] + aggp[1] + hp[...]) + bc[...]
    xv = jnp.maximum(y, 0.0) * (g[...] * GSCALE) + be[...]
    xl[...] = xv
    sr = jnp.dot(xv, Wsr[...], precision=jax.lax.Precision.HIGHEST)
    sn[...] = sr[:, 0:1]
    rn[...] = sr[:, 1:2]


def _post_call(aggp, hp, dinv, bc, g, be, Wsr):
    return pl.pallas_call(
        _post_body,
        grid=(NBLK,),
        in_specs=[
            pl.BlockSpec((NC, RBLK, H), lambda r: (0, r, 0)),
            pl.BlockSpec((RBLK, H), lambda r: (r, 0)),
            pl.BlockSpec((RBLK, 1), lambda r: (r, 0)),
            pl.BlockSpec((1, H), lambda r: (0, 0)),
            pl.BlockSpec((1, H), lambda r: (0, 0)),
            pl.BlockSpec((1, H), lambda r: (0, 0)),
            pl.BlockSpec((H, 2), lambda r: (0, 0)),
        ],
        out_specs=[
            pl.BlockSpec((RBLK, H), lambda r: (r, 0)),
            pl.BlockSpec((RBLK, 1), lambda r: (r, 0)),
            pl.BlockSpec((RBLK, 1), lambda r: (r, 0)),
        ],
        out_shape=[
            jax.ShapeDtypeStruct((NROWS, H), jnp.float32),
            jax.ShapeDtypeStruct((NROWS, 1), jnp.float32),
            jax.ShapeDtypeStruct((NROWS, 1), jnp.float32),
        ],
    )(aggp, hp, dinv, bc, g, be, Wsr)


def _pool_body(k, final, saggp, rn, bs, xl, alive, *outs):
    sc = saggp[0] + saggp[1] + bs[0, 0] + rn[...]
    u = lax.bitcast_convert_type(sc, jnp.int32)
    key = jnp.where(u < 0, (~u) ^ MINKEY, u)
    key = jnp.where(alive[...] > 0.5, key, MINKEY)

    def kth_body(i, lohi):
        lo, hi = lohi
        mid = (lo >> 1) + (hi >> 1) + (lo & hi & 1)
        cnt = jnp.sum((key >= mid).astype(jnp.int32))
        good = cnt >= k
        return jnp.where(good, mid, lo), jnp.where(good, hi, mid)

    t, _ = lax.fori_loop(0, 33, kth_body,
                         (jnp.full((), MINKEY), jnp.full((), np.int32(2**31 - 1))))
    cnt_gt = jnp.sum((key > t).astype(jnp.int32))
    tneed = k - cnt_gt
    idx = lax.broadcasted_iota(jnp.int32, key.shape, 0)
    ties = key == t

    def tie_body(i, lohi):
        lo, hi = lohi
        mid = (lo + hi) // 2
        cnt = jnp.sum((ties & (idx <= mid)).astype(jnp.int32))
        good = cnt >= tneed
        return jnp.where(good, lo, mid), jnp.where(good, mid, hi)

    _, m = lax.fori_loop(0, 15, tie_body,
                         (jnp.full((), np.int32(-1)), jnp.full((), np.int32(NROWS - 1))))
    kept = ((key > t) | (ties & (idx <= m))).astype(jnp.float32)
    tsc = jnp.tanh(sc) * kept
    xp = xl[...] * tsc
    if final:
        outs[0][...] = jnp.sum(xp, axis=0, keepdims=True) * (1.0 / float(k))
    else:
        outs[0][...] = xp
        outs[1][...] = kept


def _pool_call(k, final, saggp, rn, bs, xl, alive):
    if final:
        out_shape = [jax.ShapeDtypeStruct((1, H), jnp.float32)]
    else:
        out_shape = [jax.ShapeDtypeStruct((NROWS, H), jnp.float32),
                     jax.ShapeDtypeStruct((NROWS, 1), jnp.float32)]
    return pl.pallas_call(
        functools.partial(_pool_body, k, final),
        out_shape=out_shape,
    )(saggp, rn, bs, xl, alive)


def kernel(x, edge_index, batch,
           W1, b1, Wf1, bf1, g1, be1, Ws1, bs1, Wr1,
           W2, b2, Wf2, bf2, g2, be2, Ws2, bs2, Wr2,
           W3, b3, Wf3, bf3, g3, be3, Ws3, bs3, Wr3):
    f32 = jnp.float32
    x = x.astype(f32)
    src = edge_index[0].astype(jnp.int32).reshape(NT, E // NT)
    dst = edge_index[1].astype(jnp.int32).reshape(NT, E // NT)
    npad = CAPW - E // NT
    t_i = jnp.arange(NT, dtype=jnp.int32)[:, None]
    j_i = jnp.arange(npad, dtype=jnp.int32)[None, :]
    pad_s = (t_i * 313 + j_i) % NND
    pad_d = NND + (t_i * 37 + j_i) % NTRASH
    srcC = jnp.concatenate([src, pad_s], axis=1).reshape(-1)
    dstC = jnp.concatenate([dst, pad_d], axis=1).reshape(-1)

    xp = jnp.zeros((NROWS, x.shape[1]), f32).at[:NND].set(x)
    alive = (jnp.arange(NROWS) < NND).astype(f32).reshape(NROWS, 1)
    zer1 = jnp.zeros((NROWS,), f32)
    zer2 = jnp.zeros((NROWS, H), f32)

    Wc1, bc1, Wc2, bc2, Wc3, bc3 = _prep_call(
        W1, Wf1, b1.reshape(1, H), bf1.reshape(1, H),
        W2, Wf2, b2.reshape(1, H), bf2.reshape(1, H),
        W3, Wf3, b3.reshape(1, H), bf3.reshape(1, H))

    layer_params = [
        (Wc1, bc1, g1, be1, Ws1, bs1, Wr1, 5000),
        (Wc2, bc2, g2, be2, Ws2, bs2, Wr2, 2500),
        (Wc3, bc3, g3, be3, Ws3, bs3, Wr3, 1250),
    ]

    degp = _deg_call(dstC, zer1)
    for li, (Wc, bc, g, be, Ws, bs, Wr, k) in enumerate(layer_params):
        final = li == 2
        hp, dinv = _lin_call(xp, Wc, degp.reshape(NC, NROWS, 1))
        aggp = _agg_call(hp, srcC, dstC, zer2)
        xl, sn, rn = _post_call(aggp, hp, dinv, bc, g.reshape(1, H),
                                be.reshape(1, H),
                                jnp.concatenate([Ws, Wr], axis=1))
        saggp = _ssc_call(sn.reshape(NROWS), srcC, dstC, zer1)
        outs = _pool_call(k, final, saggp.reshape(NC, NROWS, 1), rn,
                          bs.reshape(1, 1), xl, alive)
        if final:
            return outs[0]
        xp, kept = outs
        srcC, dstC, degp = _edg_call(kept.reshape(NROWS), srcC, dstC, zer1)
        alive = kept


# speedup vs baseline: 62.5345x; 1.0512x over previous
import functools
import numpy as np
import jax
import jax.numpy as jnp
from jax import lax
from jax.experimental import pallas as pl
from jax.experimental.pallas import tpu as pltpu
from jax.experimental.pallas import tpu_sc as plsc

NND = 10000
NROWS = 10240
NTRASH = NROWS - NND
E = 640000
H = 128
NC, NS = 2, 16
NT = NC * NS
WIN = 128
CAPW = 20480
NWIN_T = CAPW // WIN
NTE = NT * CAPW
ROWS_S = NROWS // NS
GSCALE = float(1.0 / np.sqrt(1.0 + 1e-5))
MINKEY = np.int32(-2**31)

_MESH = plsc.VectorSubcoreMesh(core_axis_name="c", subcore_axis_name="s")


DGRP = 4


def _deg_body(dst_hbm, zer1_hbm, degp_hbm, didx, ones_v, acc_sh, sem_i, sem_s):
    c = lax.axis_index("c")
    s = lax.axis_index("s")
    wid = s * NC + c
    for i in range(WIN // 16):
        ones_v[pl.ds(i * 16, 16)] = jnp.ones((16,), jnp.float32)
    pltpu.sync_copy(zer1_hbm.at[pl.ds(s * ROWS_S, ROWS_S)],
                    acc_sh.at[pl.ds(s * ROWS_S, ROWS_S)])
    plsc.subcore_barrier()

    def body(i, carry):
        base = wid * CAPW + i * (DGRP * WIN)
        cps = [pltpu.async_copy(dst_hbm.at[pl.ds(base + j * WIN, WIN)],
                                didx.at[j], sem_i) for j in range(DGRP)]
        sc_ = []
        for j in range(DGRP):
            cps[j].wait()
            sc_.append(pltpu.async_copy(ones_v, acc_sh.at[didx.at[j]],
                                        sem_s, add=True))
        for cp in sc_:
            cp.wait()
        return carry

    lax.fori_loop(0, NWIN_T // DGRP, body, 0)
    plsc.subcore_barrier()
    pltpu.sync_copy(acc_sh.at[pl.ds(s * ROWS_S, ROWS_S)],
                    degp_hbm.at[c, pl.ds(s * ROWS_S, ROWS_S)])


_deg_call = pl.kernel(
    _deg_body,
    out_type=jax.ShapeDtypeStruct((NC, NROWS), jnp.float32),
    mesh=_MESH,
    scratch_types=[
        pltpu.VMEM((DGRP, WIN), jnp.int32),
        pltpu.VMEM((WIN,), jnp.float32),
        pltpu.VMEM_SHARED((NROWS,), jnp.float32),
        pltpu.SemaphoreType.DMA,
        pltpu.SemaphoreType.DMA,
    ],
)


AGRP = 2


NGRP_A = NWIN_T // AGRP


def _agg_body(hp_hbm, src_hbm, dst_hbm, zer2_hbm, out_hbm,
              sidx, didx, rows, acc_sh, sem_i, sem_g, sem_s):
    c = lax.axis_index("c")
    s = lax.axis_index("s")
    wid = s * NC + c
    pltpu.sync_copy(zer2_hbm.at[pl.ds(s * ROWS_S, ROWS_S)],
                    acc_sh.at[pl.ds(s * ROWS_S, ROWS_S)])
    plsc.subcore_barrier()

    def load_idx(g, p):
        base = wid * CAPW + g * (AGRP * WIN)
        cp0 = pltpu.async_copy(src_hbm.at[pl.ds(base, AGRP * WIN)],
                               sidx.at[p], sem_i)
        cps = [pltpu.async_copy(dst_hbm.at[pl.ds(base + j * WIN, WIN)],
                                didx.at[p, j], sem_i) for j in range(AGRP)]
        return [cp0] + cps

    def proc(p):
        gs = [pltpu.async_copy(
            hp_hbm.at[sidx.at[p, pl.ds(j * WIN, WIN)]],
            rows.at[j], sem_g) for j in range(AGRP)]
        ss = []
        for j in range(AGRP):
            gs[j].wait()
            ss.append(pltpu.async_copy(rows.at[j], acc_sh.at[didx.at[p, j]],
                                       sem_s, add=True))
        for cp in ss:
            cp.wait()

    for cp in load_idx(0, 0):
        cp.wait()

    def body(i, carry):
        lb = load_idx(2 * i + 1, 1)
        proc(0)
        for cp in lb:
            cp.wait()
        la = load_idx(jnp.minimum(2 * i + 2, NGRP_A - 1), 0)
        proc(1)
        for cp in la:
            cp.wait()
        return carry

    lax.fori_loop(0, NGRP_A // 2, body, 0)
    plsc.subcore_barrier()
    pltpu.sync_copy(acc_sh.at[pl.ds(s * ROWS_S, ROWS_S)],
                    out_hbm.at[c, pl.ds(s * ROWS_S, ROWS_S)])


_agg_call = pl.kernel(
    _agg_body,
    out_type=jax.ShapeDtypeStruct((NC, NROWS, H), jnp.float32),
    mesh=_MESH,
    scratch_types=[
        pltpu.VMEM((2, AGRP * WIN), jnp.int32),
        pltpu.VMEM((2, AGRP, WIN), jnp.int32),
        pltpu.VMEM((AGRP, WIN, H), jnp.float32),
        pltpu.VMEM_SHARED((NROWS, H), jnp.float32),
        pltpu.SemaphoreType.DMA,
        pltpu.SemaphoreType.DMA,
        pltpu.SemaphoreType.DMA,
    ],
)


SGRP = 4


def _ssc_body(sn_hbm, src_hbm, dst_hbm, zer1_hbm, out_hbm,
              sidx, didx, vals, acc_sh, sem_i, sem_g, sem_s):
    c = lax.axis_index("c")
    s = lax.axis_index("s")
    wid = s * NC + c
    pltpu.sync_copy(zer1_hbm.at[pl.ds(s * ROWS_S, ROWS_S)],
                    acc_sh.at[pl.ds(s * ROWS_S, ROWS_S)])
    plsc.subcore_barrier()

    def body(i, carry):
        base = wid * CAPW + i * (SGRP * WIN)
        cp0 = pltpu.async_copy(src_hbm.at[pl.ds(base, SGRP * WIN)], sidx, sem_i)
        cps = [pltpu.async_copy(dst_hbm.at[pl.ds(base + j * WIN, WIN)],
                                didx.at[j], sem_i) for j in range(SGRP)]
        cp0.wait()
        gs = [pltpu.async_copy(sn_hbm.at[sidx.at[pl.ds(j * WIN, WIN)]],
                               vals.at[j], sem_g) for j in range(SGRP)]
        ss = []
        for j in range(SGRP):
            cps[j].wait()
            gs[j].wait()
            ss.append(pltpu.async_copy(vals.at[j], acc_sh.at[didx.at[j]],
                                       sem_s, add=True))
        for cp in ss:
            cp.wait()
        return carry

    lax.fori_loop(0, NWIN_T // SGRP, body, 0)
    plsc.subcore_barrier()
    pltpu.sync_copy(acc_sh.at[pl.ds(s * ROWS_S, ROWS_S)],
                    out_hbm.at[c, pl.ds(s * ROWS_S, ROWS_S)])


_ssc_call = pl.kernel(
    _ssc_body,
    out_type=jax.ShapeDtypeStruct((NC, NROWS), jnp.float32),
    mesh=_MESH,
    scratch_types=[
        pltpu.VMEM((SGRP * WIN,), jnp.int32),
        pltpu.VMEM((SGRP, WIN), jnp.int32),
        pltpu.VMEM((SGRP, WIN), jnp.float32),
        pltpu.VMEM_SHARED((NROWS,), jnp.float32),
        pltpu.SemaphoreType.DMA,
        pltpu.SemaphoreType.DMA,
        pltpu.SemaphoreType.DMA,
    ],
)


EGRP = 4


def _edg_body(kept_hbm, src_hbm, dst_hbm, zer1_hbm, srco_hbm, dsto_hbm,
              degp_hbm, sbuf, dbuf, dgid, ksv, kdv, ones_v, acc_sh,
              sem_i, sem_g, sem_s):
    c = lax.axis_index("c")
    s = lax.axis_index("s")
    wid = s * NC + c
    lane = lax.iota(jnp.int32, 16)
    for n in range(WIN // 16):
        ones_v[pl.ds(n * 16, 16)] = jnp.ones((16,), jnp.float32)
    pltpu.sync_copy(zer1_hbm.at[pl.ds(s * ROWS_S, ROWS_S)],
                    acc_sh.at[pl.ds(s * ROWS_S, ROWS_S)])
    plsc.subcore_barrier()

    def body(i, carry):
        base = wid * CAPW + i * (EGRP * WIN)
        cp0 = pltpu.async_copy(src_hbm.at[pl.ds(base, EGRP * WIN)], sbuf, sem_i)
        cp1 = pltpu.async_copy(dst_hbm.at[pl.ds(base, EGRP * WIN)], dbuf, sem_i)
        cp0.wait()
        gs = [pltpu.async_copy(kept_hbm.at[sbuf.at[pl.ds(j * WIN, WIN)]],
                               ksv.at[j], sem_g) for j in range(EGRP)]
        cp1.wait()

        def pre_body(q, carry2):
            off = q * 16
            d_v = dbuf[pl.ds(off, 16)]
            spread = wid * 577 + i * 131 + off + lane
            dgid[pl.ds(off, 16)] = jnp.where(d_v < NND, d_v, spread % NND)
            return carry2

        lax.fori_loop(0, (EGRP * WIN) // 16, pre_body, 0)
        gd = [pltpu.async_copy(kept_hbm.at[dgid.at[pl.ds(j * WIN, WIN)]],
                               kdv.at[j], sem_g) for j in range(EGRP)]
        for cp in gs + gd:
            cp.wait()

        for j in range(EGRP):
            def grp_body(q, carry2, j=j):
                off = j * WIN + q * 16
                s_v = sbuf[pl.ds(off, 16)]
                d_v = dbuf[pl.ds(off, 16)]
                ks = ksv[j, pl.ds(q * 16, 16)]
                kd = kdv[j, pl.ds(q * 16, 16)]
                live = ((ks * kd) > 0.5) & (d_v < NND)
                spread = wid * 577 + i * 131 + off + lane
                sbuf[pl.ds(off, 16)] = jnp.where(live, s_v, spread % NND)
                dbuf[pl.ds(off, 16)] = jnp.where(live, d_v,
                                                 NND + (spread % NTRASH))
                return carry2

            lax.fori_loop(0, WIN // 16, grp_body, 0)
        cp2 = pltpu.async_copy(sbuf, srco_hbm.at[pl.ds(base, EGRP * WIN)], sem_i)
        cp3 = pltpu.async_copy(dbuf, dsto_hbm.at[pl.ds(base, EGRP * WIN)], sem_i)
        ds_ = [pltpu.async_copy(ones_v, acc_sh.at[dbuf.at[pl.ds(j * WIN, WIN)]],
                                sem_s, add=True) for j in range(EGRP)]
        cp2.wait()
        cp3.wait()
        for cp in ds_:
            cp.wait()
        return carry

    lax.fori_loop(0, NWIN_T // EGRP, body, 0)
    plsc.subcore_barrier()
    pltpu.sync_copy(acc_sh.at[pl.ds(s * ROWS_S, ROWS_S)],
                    degp_hbm.at[c, pl.ds(s * ROWS_S, ROWS_S)])


_edg_call = pl.kernel(
    _edg_body,
    out_type=[
        jax.ShapeDtypeStruct((NTE,), jnp.int32),
        jax.ShapeDtypeStruct((NTE,), jnp.int32),
        jax.ShapeDtypeStruct((NC, NROWS), jnp.float32),
    ],
    mesh=_MESH,
    scratch_types=[
        pltpu.VMEM((EGRP * WIN,), jnp.int32),
        pltpu.VMEM((EGRP * WIN,), jnp.int32),
        pltpu.VMEM((EGRP * WIN,), jnp.int32),
        pltpu.VMEM((EGRP, WIN), jnp.float32),
        pltpu.VMEM((EGRP, WIN), jnp.float32),
        pltpu.VMEM((WIN,), jnp.float32),
        pltpu.VMEM_SHARED((NROWS,), jnp.float32),
        pltpu.SemaphoreType.DMA,
        pltpu.SemaphoreType.DMA,
        pltpu.SemaphoreType.DMA,
    ],
)


def _prep_body(W1, Wf1, b1, bf1, W2, Wf2, b2, bf2, W3, Wf3, b3, bf3,
               Wc1, bc1, Wc2, bc2, Wc3, bc3):
    hi = jax.lax.Precision.HIGHEST
    Wc1[...] = jnp.dot(W1[...], Wf1[...], precision=hi)
    bc1[...] = jnp.dot(b1[...], Wf1[...], precision=hi) + bf1[...]
    Wc2[...] = jnp.dot(W2[...], Wf2[...], precision=hi)
    bc2[...] = jnp.dot(b2[...], Wf2[...], precision=hi) + bf2[...]
    Wc3[...] = jnp.dot(W3[...], Wf3[...], precision=hi)
    bc3[...] = jnp.dot(b3[...], Wf3[...], precision=hi) + bf3[...]


def _prep_call(W1, Wf1, b1, bf1, W2, Wf2, b2, bf2, W3, Wf3, b3, bf3):
    DIN = W1.shape[0]
    return pl.pallas_call(
        _prep_body,
        out_shape=[
            jax.ShapeDtypeStruct((DIN, H), jnp.float32),
            jax.ShapeDtypeStruct((1, H), jnp.float32),
            jax.ShapeDtypeStruct((H, H), jnp.float32),
            jax.ShapeDtypeStruct((1, H), jnp.float32),
            jax.ShapeDtypeStruct((H, H), jnp.float32),
            jax.ShapeDtypeStruct((1, H), jnp.float32),
        ],
    )(W1, Wf1, b1, bf1, W2, Wf2, b2, bf2, W3, Wf3, b3, bf3)


NBLK = 8
RBLK = NROWS // NBLK


def _lin_body(x, Wc, degp, hp, dinv):
    d = degp[0] + degp[1] + 1.0
    di = lax.rsqrt(d)
    h = jnp.dot(x[...], Wc[...], precision=jax.lax.Precision.HIGHEST)
    hp[...] = h * di
    dinv[...] = di


def _lin_call(x, Wc, degp):
    DIN = x.shape[1]
    return pl.pallas_call(
        _lin_body,
        grid=(NBLK,),
        in_specs=[
            pl.BlockSpec((RBLK, DIN), lambda r: (r, 0)),
            pl.BlockSpec((DIN, H), lambda r: (0, 0)),
            pl.BlockSpec((NC, RBLK, 1), lambda r: (0, r, 0)),
        ],
        out_specs=[
            pl.BlockSpec((RBLK, H), lambda r: (r, 0)),
            pl.BlockSpec((RBLK, 1), lambda r: (r, 0)),
        ],
        out_shape=[
            jax.ShapeDtypeStruct((NROWS, H), jnp.float32),
            jax.ShapeDtypeStruct((NROWS, 1), jnp.float32),
        ],
    )(x, Wc, degp)


def _post_body(aggp, hp, dinv, bc, g, be, Wsr, xl, sn, rn):
    y = dinv[...] * (aggp[0] + aggp[1] + hp[...]) + bc[...]
    xv = jnp.maximum(y, 0.0) * (g[...] * GSCALE) + be[...]
    xl[...] = xv
    sr = jnp.dot(xv, Wsr[...], precision=jax.lax.Precision.HIGHEST)
    sn[...] = sr[:, 0:1]
    rn[...] = sr[:, 1:2]


def _post_call(aggp, hp, dinv, bc, g, be, Wsr):
    return pl.pallas_call(
        _post_body,
        grid=(NBLK,),
        in_specs=[
            pl.BlockSpec((NC, RBLK, H), lambda r: (0, r, 0)),
            pl.BlockSpec((RBLK, H), lambda r: (r, 0)),
            pl.BlockSpec((RBLK, 1), lambda r: (r, 0)),
            pl.BlockSpec((1, H), lambda r: (0, 0)),
            pl.BlockSpec((1, H), lambda r: (0, 0)),
            pl.BlockSpec((1, H), lambda r: (0, 0)),
            pl.BlockSpec((H, 2), lambda r: (0, 0)),
        ],
        out_specs=[
            pl.BlockSpec((RBLK, H), lambda r: (r, 0)),
            pl.BlockSpec((RBLK, 1), lambda r: (r, 0)),
            pl.BlockSpec((RBLK, 1), lambda r: (r, 0)),
        ],
        out_shape=[
            jax.ShapeDtypeStruct((NROWS, H), jnp.float32),
            jax.ShapeDtypeStruct((NROWS, 1), jnp.float32),
            jax.ShapeDtypeStruct((NROWS, 1), jnp.float32),
        ],
    )(aggp, hp, dinv, bc, g, be, Wsr)


def _pool_body(k, final, saggp, rn, bs, xl, alive, *outs):
    sc = saggp[0] + saggp[1] + bs[0, 0] + rn[...]
    u = lax.bitcast_convert_type(sc, jnp.int32)
    key = jnp.where(u < 0, (~u) ^ MINKEY, u)
    key = jnp.where(alive[...] > 0.5, key, MINKEY)

    def kth_body(i, lohi):
        lo, hi = lohi
        mid = (lo >> 1) + (hi >> 1) + (lo & hi & 1)
        cnt = jnp.sum((key >= mid).astype(jnp.int32))
        good = cnt >= k
        return jnp.where(good, mid, lo), jnp.where(good, hi, mid)

    t, _ = lax.fori_loop(0, 33, kth_body,
                         (jnp.full((), MINKEY), jnp.full((), np.int32(2**31 - 1))))
    cnt_gt = jnp.sum((key > t).astype(jnp.int32))
    tneed = k - cnt_gt
    idx = lax.broadcasted_iota(jnp.int32, key.shape, 0)
    ties = key == t

    def tie_body(i, lohi):
        lo, hi = lohi
        mid = (lo + hi) // 2
        cnt = jnp.sum((ties & (idx <= mid)).astype(jnp.int32))
        good = cnt >= tneed
        return jnp.where(good, lo, mid), jnp.where(good, mid, hi)

    _, m = lax.fori_loop(0, 15, tie_body,
                         (jnp.full((), np.int32(-1)), jnp.full((), np.int32(NROWS - 1))))
    kept = ((key > t) | (ties & (idx <= m))).astype(jnp.float32)
    tsc = jnp.tanh(sc) * kept
    xp = xl[...] * tsc
    if final:
        outs[0][...] = jnp.sum(xp, axis=0, keepdims=True) * (1.0 / float(k))
    else:
        outs[0][...] = xp
        outs[1][...] = kept


def _pool_call(k, final, saggp, rn, bs, xl, alive):
    if final:
        out_shape = [jax.ShapeDtypeStruct((1, H), jnp.float32)]
    else:
        out_shape = [jax.ShapeDtypeStruct((NROWS, H), jnp.float32),
                     jax.ShapeDtypeStruct((NROWS, 1), jnp.float32)]
    return pl.pallas_call(
        functools.partial(_pool_body, k, final),
        out_shape=out_shape,
    )(saggp, rn, bs, xl, alive)


def kernel(x, edge_index, batch,
           W1, b1, Wf1, bf1, g1, be1, Ws1, bs1, Wr1,
           W2, b2, Wf2, bf2, g2, be2, Ws2, bs2, Wr2,
           W3, b3, Wf3, bf3, g3, be3, Ws3, bs3, Wr3):
    f32 = jnp.float32
    x = x.astype(f32)
    src = edge_index[0].astype(jnp.int32).reshape(NT, E // NT)
    dst = edge_index[1].astype(jnp.int32).reshape(NT, E // NT)
    npad = CAPW - E // NT
    t_i = jnp.arange(NT, dtype=jnp.int32)[:, None]
    j_i = jnp.arange(npad, dtype=jnp.int32)[None, :]
    pad_s = (t_i * 313 + j_i) % NND
    pad_d = NND + (t_i * 37 + j_i) % NTRASH
    srcC = jnp.concatenate([src, pad_s], axis=1).reshape(-1)
    dstC = jnp.concatenate([dst, pad_d], axis=1).reshape(-1)

    xp = jnp.zeros((NROWS, x.shape[1]), f32).at[:NND].set(x)
    alive = (jnp.arange(NROWS) < NND).astype(f32).reshape(NROWS, 1)
    zer1 = jnp.zeros((NROWS,), f32)
    zer2 = jnp.zeros((NROWS, H), f32)

    Wc1, bc1, Wc2, bc2, Wc3, bc3 = _prep_call(
        W1, Wf1, b1.reshape(1, H), bf1.reshape(1, H),
        W2, Wf2, b2.reshape(1, H), bf2.reshape(1, H),
        W3, Wf3, b3.reshape(1, H), bf3.reshape(1, H))

    layer_params = [
        (Wc1, bc1, g1, be1, Ws1, bs1, Wr1, 5000),
        (Wc2, bc2, g2, be2, Ws2, bs2, Wr2, 2500),
        (Wc3, bc3, g3, be3, Ws3, bs3, Wr3, 1250),
    ]

    degp = _deg_call(dstC, zer1)
    for li, (Wc, bc, g, be, Ws, bs, Wr, k) in enumerate(layer_params):
        final = li == 2
        hp, dinv = _lin_call(xp, Wc, degp.reshape(NC, NROWS, 1))
        aggp = _agg_call(hp, srcC, dstC, zer2)
        xl, sn, rn = _post_call(aggp, hp, dinv, bc, g.reshape(1, H),
                                be.reshape(1, H),
                                jnp.concatenate([Ws, Wr], axis=1))
        saggp = _ssc_call(sn.reshape(NROWS), srcC, dstC, zer1)
        outs = _pool_call(k, final, saggp.reshape(NC, NROWS, 1), rn,
                          bs.reshape(1, 1), xl, alive)
        if final:
            return outs[0]
        xp, kept = outs
        srcC, dstC, degp = _edg_call(kept.reshape(NROWS), srcC, dstC, zer1)
        alive = kept


# speedup vs baseline: 63.5876x; 1.0168x over previous
import functools
import numpy as np
import jax
import jax.numpy as jnp
from jax import lax
from jax.experimental import pallas as pl
from jax.experimental.pallas import tpu as pltpu
from jax.experimental.pallas import tpu_sc as plsc

NND = 10000
NROWS = 10240
NTRASH = NROWS - NND
E = 640000
H = 128
NC, NS = 2, 16
NT = NC * NS
WIN = 128
CAPW = 20480
NWIN_T = CAPW // WIN
NTE = NT * CAPW
ROWS_S = NROWS // NS
GSCALE = float(1.0 / np.sqrt(1.0 + 1e-5))
MINKEY = np.int32(-2**31)

_MESH = plsc.VectorSubcoreMesh(core_axis_name="c", subcore_axis_name="s")


DGRP = 4


def _deg_body(dst_hbm, zer1_hbm, degp_hbm, didx, ones_v, acc_sh, sem_i, sem_s):
    c = lax.axis_index("c")
    s = lax.axis_index("s")
    wid = s * NC + c
    for i in range(WIN // 16):
        ones_v[pl.ds(i * 16, 16)] = jnp.ones((16,), jnp.float32)
    pltpu.sync_copy(zer1_hbm.at[pl.ds(s * ROWS_S, ROWS_S)],
                    acc_sh.at[pl.ds(s * ROWS_S, ROWS_S)])
    plsc.subcore_barrier()

    def body(i, carry):
        base = wid * CAPW + i * (DGRP * WIN)
        cps = [pltpu.async_copy(dst_hbm.at[pl.ds(base + j * WIN, WIN)],
                                didx.at[j], sem_i) for j in range(DGRP)]
        sc_ = []
        for j in range(DGRP):
            cps[j].wait()
            sc_.append(pltpu.async_copy(ones_v, acc_sh.at[didx.at[j]],
                                        sem_s, add=True))
        for cp in sc_:
            cp.wait()
        return carry

    lax.fori_loop(0, NWIN_T // DGRP, body, 0)
    plsc.subcore_barrier()
    pltpu.sync_copy(acc_sh.at[pl.ds(s * ROWS_S, ROWS_S)],
                    degp_hbm.at[c, pl.ds(s * ROWS_S, ROWS_S)])


_deg_call = pl.kernel(
    _deg_body,
    out_type=jax.ShapeDtypeStruct((NC, NROWS), jnp.float32),
    mesh=_MESH,
    scratch_types=[
        pltpu.VMEM((DGRP, WIN), jnp.int32),
        pltpu.VMEM((WIN,), jnp.float32),
        pltpu.VMEM_SHARED((NROWS,), jnp.float32),
        pltpu.SemaphoreType.DMA,
        pltpu.SemaphoreType.DMA,
    ],
)


AGRP = 2


NGRP_A = NWIN_T // AGRP


def _agg_body(hp_hbm, src_hbm, dst_hbm, zer2_hbm, out_hbm,
              sidx, didx, rows, acc_sh, sem_i, sem_g, sem_s):
    c = lax.axis_index("c")
    s = lax.axis_index("s")
    wid = s * NC + c
    pltpu.sync_copy(zer2_hbm.at[pl.ds(s * ROWS_S, ROWS_S)],
                    acc_sh.at[pl.ds(s * ROWS_S, ROWS_S)])
    plsc.subcore_barrier()

    def load_idx(g, p):
        base = wid * CAPW + g * (AGRP * WIN)
        cp0 = pltpu.async_copy(src_hbm.at[pl.ds(base, AGRP * WIN)],
                               sidx.at[p], sem_i)
        cps = [pltpu.async_copy(dst_hbm.at[pl.ds(base + j * WIN, WIN)],
                                didx.at[p, j], sem_i) for j in range(AGRP)]
        return [cp0] + cps

    def proc(p):
        gs = [pltpu.async_copy(
            hp_hbm.at[sidx.at[p, pl.ds(j * WIN, WIN)]],
            rows.at[j], sem_g) for j in range(AGRP)]
        ss = []
        for j in range(AGRP):
            gs[j].wait()
            ss.append(pltpu.async_copy(rows.at[j], acc_sh.at[didx.at[p, j]],
                                       sem_s, add=True))
        for cp in ss:
            cp.wait()

    for cp in load_idx(0, 0):
        cp.wait()

    def body(i, carry):
        lb = load_idx(2 * i + 1, 1)
        proc(0)
        for cp in lb:
            cp.wait()
        la = load_idx(jnp.minimum(2 * i + 2, NGRP_A - 1), 0)
        proc(1)
        for cp in la:
            cp.wait()
        return carry

    lax.fori_loop(0, NGRP_A // 2, body, 0)
    plsc.subcore_barrier()
    pltpu.sync_copy(acc_sh.at[pl.ds(s * ROWS_S, ROWS_S)],
                    out_hbm.at[c, pl.ds(s * ROWS_S, ROWS_S)])


_agg_call = pl.kernel(
    _agg_body,
    out_type=jax.ShapeDtypeStruct((NC, NROWS, H), jnp.float32),
    mesh=_MESH,
    scratch_types=[
        pltpu.VMEM((2, AGRP * WIN), jnp.int32),
        pltpu.VMEM((2, AGRP, WIN), jnp.int32),
        pltpu.VMEM((AGRP, WIN, H), jnp.float32),
        pltpu.VMEM_SHARED((NROWS, H), jnp.float32),
        pltpu.SemaphoreType.DMA,
        pltpu.SemaphoreType.DMA,
        pltpu.SemaphoreType.DMA,
    ],
)


SGRP = 4


NGRP_S = NWIN_T // SGRP


def _ssc_body(sn_hbm, src_hbm, dst_hbm, zer1_hbm, out_hbm,
              sidx, didx, vals, acc_sh, sem_i, sem_g, sem_s):
    c = lax.axis_index("c")
    s = lax.axis_index("s")
    wid = s * NC + c
    pltpu.sync_copy(zer1_hbm.at[pl.ds(s * ROWS_S, ROWS_S)],
                    acc_sh.at[pl.ds(s * ROWS_S, ROWS_S)])
    plsc.subcore_barrier()

    def load_idx(g, p):
        base = wid * CAPW + g * (SGRP * WIN)
        cp0 = pltpu.async_copy(src_hbm.at[pl.ds(base, SGRP * WIN)],
                               sidx.at[p], sem_i)
        cps = [pltpu.async_copy(dst_hbm.at[pl.ds(base + j * WIN, WIN)],
                                didx.at[p, j], sem_i) for j in range(SGRP)]
        return [cp0] + cps

    def proc(p):
        gs = [pltpu.async_copy(
            sn_hbm.at[sidx.at[p, pl.ds(j * WIN, WIN)]],
            vals.at[j], sem_g) for j in range(SGRP)]
        ss = []
        for j in range(SGRP):
            gs[j].wait()
            ss.append(pltpu.async_copy(vals.at[j], acc_sh.at[didx.at[p, j]],
                                       sem_s, add=True))
        for cp in ss:
            cp.wait()

    for cp in load_idx(0, 0):
        cp.wait()

    def body(i, carry):
        lb = load_idx(2 * i + 1, 1)
        proc(0)
        for cp in lb:
            cp.wait()
        la = load_idx(jnp.minimum(2 * i + 2, NGRP_S - 1), 0)
        proc(1)
        for cp in la:
            cp.wait()
        return carry

    lax.fori_loop(0, NGRP_S // 2, body, 0)
    plsc.subcore_barrier()
    pltpu.sync_copy(acc_sh.at[pl.ds(s * ROWS_S, ROWS_S)],
                    out_hbm.at[c, pl.ds(s * ROWS_S, ROWS_S)])


_ssc_call = pl.kernel(
    _ssc_body,
    out_type=jax.ShapeDtypeStruct((NC, NROWS), jnp.float32),
    mesh=_MESH,
    scratch_types=[
        pltpu.VMEM((2, SGRP * WIN), jnp.int32),
        pltpu.VMEM((2, SGRP, WIN), jnp.int32),
        pltpu.VMEM((SGRP, WIN), jnp.float32),
        pltpu.VMEM_SHARED((NROWS,), jnp.float32),
        pltpu.SemaphoreType.DMA,
        pltpu.SemaphoreType.DMA,
        pltpu.SemaphoreType.DMA,
    ],
)


EGRP = 4


def _edg_body(kept_hbm, src_hbm, dst_hbm, zer1_hbm, srco_hbm, dsto_hbm,
              degp_hbm, sbuf, dbuf, dgid, ksv, kdv, ones_v, acc_sh,
              sem_i, sem_g, sem_s):
    c = lax.axis_index("c")
    s = lax.axis_index("s")
    wid = s * NC + c
    lane = lax.iota(jnp.int32, 16)
    for n in range(WIN // 16):
        ones_v[pl.ds(n * 16, 16)] = jnp.ones((16,), jnp.float32)
    pltpu.sync_copy(zer1_hbm.at[pl.ds(s * ROWS_S, ROWS_S)],
                    acc_sh.at[pl.ds(s * ROWS_S, ROWS_S)])
    plsc.subcore_barrier()

    def body(i, carry):
        base = wid * CAPW + i * (EGRP * WIN)
        cp0 = pltpu.async_copy(src_hbm.at[pl.ds(base, EGRP * WIN)], sbuf, sem_i)
        cp1 = pltpu.async_copy(dst_hbm.at[pl.ds(base, EGRP * WIN)], dbuf, sem_i)
        cp0.wait()
        gs = [pltpu.async_copy(kept_hbm.at[sbuf.at[pl.ds(j * WIN, WIN)]],
                               ksv.at[j], sem_g) for j in range(EGRP)]
        cp1.wait()

        def pre_body(q, carry2):
            off = q * 16
            d_v = dbuf[pl.ds(off, 16)]
            spread = wid * 577 + i * 131 + off + lane
            dgid[pl.ds(off, 16)] = jnp.where(d_v < NND, d_v, spread % NND)
            return carry2

        lax.fori_loop(0, (EGRP * WIN) // 16, pre_body, 0)
        gd = [pltpu.async_copy(kept_hbm.at[dgid.at[pl.ds(j * WIN, WIN)]],
                               kdv.at[j], sem_g) for j in range(EGRP)]
        for cp in gs + gd:
            cp.wait()

        for j in range(EGRP):
            def grp_body(q, carry2, j=j):
                off = j * WIN + q * 16
                s_v = sbuf[pl.ds(off, 16)]
                d_v = dbuf[pl.ds(off, 16)]
                ks = ksv[j, pl.ds(q * 16, 16)]
                kd = kdv[j, pl.ds(q * 16, 16)]
                live = ((ks * kd) > 0.5) & (d_v < NND)
                spread = wid * 577 + i * 131 + off + lane
                sbuf[pl.ds(off, 16)] = jnp.where(live, s_v, spread % NND)
                dbuf[pl.ds(off, 16)] = jnp.where(live, d_v,
                                                 NND + (spread % NTRASH))
                return carry2

            lax.fori_loop(0, WIN // 16, grp_body, 0)
        cp2 = pltpu.async_copy(sbuf, srco_hbm.at[pl.ds(base, EGRP * WIN)], sem_i)
        cp3 = pltpu.async_copy(dbuf, dsto_hbm.at[pl.ds(base, EGRP * WIN)], sem_i)
        ds_ = [pltpu.async_copy(ones_v, acc_sh.at[dbuf.at[pl.ds(j * WIN, WIN)]],
                                sem_s, add=True) for j in range(EGRP)]
        cp2.wait()
        cp3.wait()
        for cp in ds_:
            cp.wait()
        return carry

    lax.fori_loop(0, NWIN_T // EGRP, body, 0)
    plsc.subcore_barrier()
    pltpu.sync_copy(acc_sh.at[pl.ds(s * ROWS_S, ROWS_S)],
                    degp_hbm.at[c, pl.ds(s * ROWS_S, ROWS_S)])


_edg_call = pl.kernel(
    _edg_body,
    out_type=[
        jax.ShapeDtypeStruct((NTE,), jnp.int32),
        jax.ShapeDtypeStruct((NTE,), jnp.int32),
        jax.ShapeDtypeStruct((NC, NROWS), jnp.float32),
    ],
    mesh=_MESH,
    scratch_types=[
        pltpu.VMEM((EGRP * WIN,), jnp.int32),
        pltpu.VMEM((EGRP * WIN,), jnp.int32),
        pltpu.VMEM((EGRP * WIN,), jnp.int32),
        pltpu.VMEM((EGRP, WIN), jnp.float32),
        pltpu.VMEM((EGRP, WIN), jnp.float32),
        pltpu.VMEM((WIN,), jnp.float32),
        pltpu.VMEM_SHARED((NROWS,), jnp.float32),
        pltpu.SemaphoreType.DMA,
        pltpu.SemaphoreType.DMA,
        pltpu.SemaphoreType.DMA,
    ],
)


def _prep_body(W1, Wf1, b1, bf1, W2, Wf2, b2, bf2, W3, Wf3, b3, bf3,
               Wc1, bc1, Wc2, bc2, Wc3, bc3):
    hi = jax.lax.Precision.HIGHEST
    Wc1[...] = jnp.dot(W1[...], Wf1[...], precision=hi)
    bc1[...] = jnp.dot(b1[...], Wf1[...], precision=hi) + bf1[...]
    Wc2[...] = jnp.dot(W2[...], Wf2[...], precision=hi)
    bc2[...] = jnp.dot(b2[...], Wf2[...], precision=hi) + bf2[...]
    Wc3[...] = jnp.dot(W3[...], Wf3[...], precision=hi)
    bc3[...] = jnp.dot(b3[...], Wf3[...], precision=hi) + bf3[...]


def _prep_call(W1, Wf1, b1, bf1, W2, Wf2, b2, bf2, W3, Wf3, b3, bf3):
    DIN = W1.shape[0]
    return pl.pallas_call(
        _prep_body,
        out_shape=[
            jax.ShapeDtypeStruct((DIN, H), jnp.float32),
            jax.ShapeDtypeStruct((1, H), jnp.float32),
            jax.ShapeDtypeStruct((H, H), jnp.float32),
            jax.ShapeDtypeStruct((1, H), jnp.float32),
            jax.ShapeDtypeStruct((H, H), jnp.float32),
            jax.ShapeDtypeStruct((1, H), jnp.float32),
        ],
    )(W1, Wf1, b1, bf1, W2, Wf2, b2, bf2, W3, Wf3, b3, bf3)


NBLK = 8
RBLK = NROWS // NBLK


def _lin_body(x, Wc, degp, hp, dinv):
    d = degp[0] + degp[1] + 1.0
    di = lax.rsqrt(d)
    h = jnp.dot(x[...], Wc[...], precision=jax.lax.Precision.HIGHEST)
    hp[...] = h * di
    dinv[...] = di


def _lin_call(x, Wc, degp):
    DIN = x.shape[1]
    return pl.pallas_call(
        _lin_body,
        grid=(NBLK,),
        in_specs=[
            pl.BlockSpec((RBLK, DIN), lambda r: (r, 0)),
            pl.BlockSpec((DIN, H), lambda r: (0, 0)),
            pl.BlockSpec((NC, RBLK, 1), lambda r: (0, r, 0)),
        ],
        out_specs=[
            pl.BlockSpec((RBLK, H), lambda r: (r, 0)),
            pl.BlockSpec((RBLK, 1), lambda r: (r, 0)),
        ],
        out_shape=[
            jax.ShapeDtypeStruct((NROWS, H), jnp.float32),
            jax.ShapeDtypeStruct((NROWS, 1), jnp.float32),
        ],
    )(x, Wc, degp)


def _post_body(aggp, hp, dinv, bc, g, be, Wsr, xl, sn, rn):
    y = dinv[...] * (aggp[0] + aggp[1] + hp[...]) + bc[...]
    xv = jnp.maximum(y, 0.0) * (g[...] * GSCALE) + be[...]
    xl[...] = xv
    sr = jnp.dot(xv, Wsr[...], precision=jax.lax.Precision.HIGHEST)
    sn[...] = sr[:, 0:1]
    rn[...] = sr[:, 1:2]


def _post_call(aggp, hp, dinv, bc, g, be, Wsr):
    return pl.pallas_call(
        _post_body,
        grid=(NBLK,),
        in_specs=[
            pl.BlockSpec((NC, RBLK, H), lambda r: (0, r, 0)),
            pl.BlockSpec((RBLK, H), lambda r: (r, 0)),
            pl.BlockSpec((RBLK, 1), lambda r: (r, 0)),
            pl.BlockSpec((1, H), lambda r: (0, 0)),
            pl.BlockSpec((1, H), lambda r: (0, 0)),
            pl.BlockSpec((1, H), lambda r: (0, 0)),
            pl.BlockSpec((H, 2), lambda r: (0, 0)),
        ],
        out_specs=[
            pl.BlockSpec((RBLK, H), lambda r: (r, 0)),
            pl.BlockSpec((RBLK, 1), lambda r: (r, 0)),
            pl.BlockSpec((RBLK, 1), lambda r: (r, 0)),
        ],
        out_shape=[
            jax.ShapeDtypeStruct((NROWS, H), jnp.float32),
            jax.ShapeDtypeStruct((NROWS, 1), jnp.float32),
            jax.ShapeDtypeStruct((NROWS, 1), jnp.float32),
        ],
    )(aggp, hp, dinv, bc, g, be, Wsr)


def _pool_body(k, final, saggp, rn, bs, xl, alive, *outs):
    sc = saggp[0] + saggp[1] + bs[0, 0] + rn[...]
    u = lax.bitcast_convert_type(sc, jnp.int32)
    key = jnp.where(u < 0, (~u) ^ MINKEY, u)
    key = jnp.where(alive[...] > 0.5, key, MINKEY)

    def kth_body(i, lohi):
        lo, hi = lohi
        mid = (lo >> 1) + (hi >> 1) + (lo & hi & 1)
        cnt = jnp.sum((key >= mid).astype(jnp.int32))
        good = cnt >= k
        return jnp.where(good, mid, lo), jnp.where(good, hi, mid)

    t, _ = lax.fori_loop(0, 33, kth_body,
                         (jnp.full((), MINKEY), jnp.full((), np.int32(2**31 - 1))))
    cnt_gt = jnp.sum((key > t).astype(jnp.int32))
    tneed = k - cnt_gt
    idx = lax.broadcasted_iota(jnp.int32, key.shape, 0)
    ties = key == t

    def tie_body(i, lohi):
        lo, hi = lohi
        mid = (lo + hi) // 2
        cnt = jnp.sum((ties & (idx <= mid)).astype(jnp.int32))
        good = cnt >= tneed
        return jnp.where(good, lo, mid), jnp.where(good, mid, hi)

    _, m = lax.fori_loop(0, 15, tie_body,
                         (jnp.full((), np.int32(-1)), jnp.full((), np.int32(NROWS - 1))))
    kept = ((key > t) | (ties & (idx <= m))).astype(jnp.float32)
    tsc = jnp.tanh(sc) * kept
    xp = xl[...] * tsc
    if final:
        outs[0][...] = jnp.sum(xp, axis=0, keepdims=True) * (1.0 / float(k))
    else:
        outs[0][...] = xp
        outs[1][...] = kept


def _pool_call(k, final, saggp, rn, bs, xl, alive):
    if final:
        out_shape = [jax.ShapeDtypeStruct((1, H), jnp.float32)]
    else:
        out_shape = [jax.ShapeDtypeStruct((NROWS, H), jnp.float32),
                     jax.ShapeDtypeStruct((NROWS, 1), jnp.float32)]
    return pl.pallas_call(
        functools.partial(_pool_body, k, final),
        out_shape=out_shape,
    )(saggp, rn, bs, xl, alive)


def kernel(x, edge_index, batch,
           W1, b1, Wf1, bf1, g1, be1, Ws1, bs1, Wr1,
           W2, b2, Wf2, bf2, g2, be2, Ws2, bs2, Wr2,
           W3, b3, Wf3, bf3, g3, be3, Ws3, bs3, Wr3):
    f32 = jnp.float32
    x = x.astype(f32)
    src = edge_index[0].astype(jnp.int32).reshape(NT, E // NT)
    dst = edge_index[1].astype(jnp.int32).reshape(NT, E // NT)
    npad = CAPW - E // NT
    t_i = jnp.arange(NT, dtype=jnp.int32)[:, None]
    j_i = jnp.arange(npad, dtype=jnp.int32)[None, :]
    pad_s = (t_i * 313 + j_i) % NND
    pad_d = NND + (t_i * 37 + j_i) % NTRASH
    srcC = jnp.concatenate([src, pad_s], axis=1).reshape(-1)
    dstC = jnp.concatenate([dst, pad_d], axis=1).reshape(-1)

    xp = jnp.zeros((NROWS, x.shape[1]), f32).at[:NND].set(x)
    alive = (jnp.arange(NROWS) < NND).astype(f32).reshape(NROWS, 1)
    zer1 = jnp.zeros((NROWS,), f32)
    zer2 = jnp.zeros((NROWS, H), f32)

    Wc1, bc1, Wc2, bc2, Wc3, bc3 = _prep_call(
        W1, Wf1, b1.reshape(1, H), bf1.reshape(1, H),
        W2, Wf2, b2.reshape(1, H), bf2.reshape(1, H),
        W3, Wf3, b3.reshape(1, H), bf3.reshape(1, H))

    layer_params = [
        (Wc1, bc1, g1, be1, Ws1, bs1, Wr1, 5000),
        (Wc2, bc2, g2, be2, Ws2, bs2, Wr2, 2500),
        (Wc3, bc3, g3, be3, Ws3, bs3, Wr3, 1250),
    ]

    degp = _deg_call(dstC, zer1)
    for li, (Wc, bc, g, be, Ws, bs, Wr, k) in enumerate(layer_params):
        final = li == 2
        hp, dinv = _lin_call(xp, Wc, degp.reshape(NC, NROWS, 1))
        aggp = _agg_call(hp, srcC, dstC, zer2)
        xl, sn, rn = _post_call(aggp, hp, dinv, bc, g.reshape(1, H),
                                be.reshape(1, H),
                                jnp.concatenate([Ws, Wr], axis=1))
        saggp = _ssc_call(sn.reshape(NROWS), srcC, dstC, zer1)
        outs = _pool_call(k, final, saggp.reshape(NC, NROWS, 1), rn,
                          bs.reshape(1, 1), xl, alive)
        if final:
            return outs[0]
        xp, kept = outs
        srcC, dstC, degp = _edg_call(kept.reshape(NROWS), srcC, dstC, zer1)
        alive = kept
